# baseline (speedup 1.0000x reference)
import functools

import jax
import jax.numpy as jnp
from jax.experimental import pallas as pl
from jax.experimental.pallas import tpu as pltpu

_BLK = 2048
_EPS = 1e-5
_F32 = jnp.float32


def _cdiv(a, b):
    return (a + b - 1) // b


def _colmask(c, n_true):
    col = pl.program_id(0) * _BLK + jax.lax.broadcasted_iota(
        jnp.int32, (c, _BLK), 1)
    return col < n_true


def _mat_stats_kernel(x_ref, w_ref, s_ref, sum_ref, ssq_ref, *, n_true):
    s = jnp.dot(w_ref[...], x_ref[...], preferred_element_type=_F32)
    s_ref[...] = s

    @pl.when(pl.program_id(0) == 0)
    def _init():
        sum_ref[...] = jnp.zeros_like(sum_ref)
        ssq_ref[...] = jnp.zeros_like(ssq_ref)

    sm = jnp.where(_colmask(s.shape[0], n_true), s, 0.0)
    sum_ref[...] += jnp.sum(sm, axis=1, keepdims=True)
    ssq_ref[...] += jnp.sum(sm * sm, axis=1, keepdims=True)


def _norm_mat_stats_kernel(s_in_ref, sc_ref, sh_ref, w_ref,
                           s_ref, sum_ref, ssq_ref, *, n_true):
    a = jnp.maximum(s_in_ref[...] * sc_ref[...] + sh_ref[...], 0.0)
    s = jnp.dot(w_ref[...], a, preferred_element_type=_F32)
    s_ref[...] = s

    @pl.when(pl.program_id(0) == 0)
    def _init():
        sum_ref[...] = jnp.zeros_like(sum_ref)
        ssq_ref[...] = jnp.zeros_like(ssq_ref)

    sm = jnp.where(_colmask(s.shape[0], n_true), s, 0.0)
    sum_ref[...] += jnp.sum(sm, axis=1, keepdims=True)
    ssq_ref[...] += jnp.sum(sm * sm, axis=1, keepdims=True)


def _norm_relu_kernel(s_ref, sc_ref, sh_ref, o_ref):
    o_ref[...] = jnp.maximum(s_ref[...] * sc_ref[...] + sh_ref[...], 0.0)


def _norm_gram_kernel(s_ref, sc_ref, sh_ref, sum_ref, gram_ref, *, n_true):
    z = jnp.maximum(s_ref[...] * sc_ref[...] + sh_ref[...], 0.0)
    z = jnp.where(_colmask(z.shape[0], n_true), z, 0.0)

    @pl.when(pl.program_id(0) == 0)
    def _init():
        sum_ref[...] = jnp.zeros_like(sum_ref)
        gram_ref[...] = jnp.zeros_like(gram_ref)

    sum_ref[...] += jnp.sum(z, axis=1, keepdims=True)
    gram_ref[...] += jax.lax.dot_general(
        z, z, (((1,), (1,)), ((), ())), preferred_element_type=_F32)


def _final_kernel(s3_ref, sc3_ref, sh3_ref, meanz_ref, cov_ref,
                  wg1_ref, wg2_ref, wg3_ref, bg2_ref, bg3_ref,
                  out_ref, w1f_ref, sh4_ref):
    @pl.when(pl.program_id(0) == 0)
    def _fold():
        wg1 = wg1_ref[...]
        m = jnp.dot(wg1, cov_ref[...], preferred_element_type=_F32)
        var4 = jnp.sum(m * wg1, axis=1, keepdims=True)
        mu4 = jnp.dot(wg1, meanz_ref[...], preferred_element_type=_F32)
        inv4 = jax.lax.rsqrt(var4 + _EPS)
        w1f_ref[...] = wg1 * inv4
        sh4_ref[...] = -mu4 * inv4

    z = jnp.maximum(s3_ref[...] * sc3_ref[...] + sh3_ref[...], 0.0)
    h1 = jnp.dot(w1f_ref[...], z, preferred_element_type=_F32) + sh4_ref[...]
    h1 = jnp.where(h1 >= 0, h1, 0.01 * h1)
    h2 = jnp.dot(wg2_ref[...], h1, preferred_element_type=_F32) + bg2_ref[...]
    h2 = jnp.where(h2 >= 0, h2, 0.01 * h2)
    out_ref[...] = (jnp.dot(wg3_ref[...], h2, preferred_element_type=_F32)
                    + bg3_ref[...])


def _affine_from_stats(sm, sq, n):
    mu = sm / n
    var = sq / n - mu * mu
    inv = jax.lax.rsqrt(var + _EPS)
    return inv, -mu * inv


def _col_spec(c):
    return pl.BlockSpec((c, _BLK), lambda i: (0, i))


def _full_spec(r, c):
    return pl.BlockSpec((r, c), lambda i: (0, 0))


def kernel(node_features, factor_features_0, nn_idx_0, etype_0,
           Wm0, bm0, Wm1, bm1, Wmp, bmp, Wg1, bg1, Wg2, bg2, Wg3, bg3):
    del nn_idx_0, etype_0, bm0, bm1, bmp, bg1
    xn = node_features[0, :, :, 0]
    xf = factor_features_0[0, :, :, 0]
    nn = xn.shape[1]
    nf = xf.shape[1]
    cp = Wmp.shape[0]
    c1 = Wg1.shape[0]
    c3 = Wg3.shape[0]
    gn = _cdiv(nn, _BLK)
    gf = _cdiv(nf, _BLK)
    nnp = gn * _BLK
    nfp = gf * _BLK

    def mat_stats(x, w, n, g, np_):
        co, ci = w.shape
        return pl.pallas_call(
            functools.partial(_mat_stats_kernel, n_true=n),
            grid=(g,),
            in_specs=[_col_spec(ci), _full_spec(co, ci)],
            out_specs=[_col_spec(co), _full_spec(co, 1), _full_spec(co, 1)],
            out_shape=[jax.ShapeDtypeStruct((co, np_), _F32),
                       jax.ShapeDtypeStruct((co, 1), _F32),
                       jax.ShapeDtypeStruct((co, 1), _F32)],
        )(x, w)

    s1, sum1, ssq1 = mat_stats(xn, Wm0, nn, gn, nnp)
    s2, sum2, ssq2 = mat_stats(xf, Wm1, nf, gf, nfp)
    sc1, sh1 = _affine_from_stats(sum1, ssq1, nn)
    sc2, sh2 = _affine_from_stats(sum2, ssq2, nf)

    def norm_mat_stats(s_in, sc, sh, w, n, g, np_):
        co, ci = w.shape
        return pl.pallas_call(
            functools.partial(_norm_mat_stats_kernel, n_true=n),
            grid=(g,),
            in_specs=[_col_spec(ci), _full_spec(ci, 1), _full_spec(ci, 1),
                      _full_spec(co, ci)],
            out_specs=[_col_spec(co), _full_spec(co, 1), _full_spec(co, 1)],
            out_shape=[jax.ShapeDtypeStruct((co, np_), _F32),
                       jax.ShapeDtypeStruct((co, 1), _F32),
                       jax.ShapeDtypeStruct((co, 1), _F32)],
        )(s_in, sc, sh, w)

    s3n, sum3n, ssq3n = norm_mat_stats(s1, sc1, sh1, Wmp, nn, gn, nnp)
    s3f, sum3f, ssq3f = norm_mat_stats(s2, sc2, sh2, Wmp, nf, gf, nfp)
    sc3, sh3 = _affine_from_stats(sum3n + sum3f, ssq3n + ssq3f, nn + nf)

    cf = pl.pallas_call(
        _norm_relu_kernel,
        grid=(gf,),
        in_specs=[_col_spec(cp), _full_spec(cp, 1), _full_spec(cp, 1)],
        out_specs=_col_spec(cp),
        out_shape=jax.ShapeDtypeStruct((cp, nf), _F32),
    )(s3f, sc3, sh3)

    sumz, gram = pl.pallas_call(
        functools.partial(_norm_gram_kernel, n_true=nn),
        grid=(gn,),
        in_specs=[_col_spec(cp), _full_spec(cp, 1), _full_spec(cp, 1)],
        out_specs=[_full_spec(cp, 1), _full_spec(cp, cp)],
        out_shape=[jax.ShapeDtypeStruct((cp, 1), _F32),
                   jax.ShapeDtypeStruct((cp, cp), _F32)],
    )(s3n, sc3, sh3)
    meanz = sumz / nn
    cov = gram / nn - meanz * meanz.T

    out = pl.pallas_call(
        _final_kernel,
        grid=(gn,),
        in_specs=[_col_spec(cp), _full_spec(cp, 1), _full_spec(cp, 1),
                  _full_spec(cp, 1), _full_spec(cp, cp),
                  _full_spec(c1, cp), _full_spec(c1, c1), _full_spec(c3, c1),
                  _full_spec(c1, 1), _full_spec(c3, 1)],
        out_specs=_col_spec(c3),
        out_shape=jax.ShapeDtypeStruct((c3, nn), _F32),
        scratch_shapes=[pltpu.VMEM((c1, cp), _F32),
                        pltpu.VMEM((c1, 1), _F32)],
    )(s3n, sc3, sh3, meanz, cov, Wg1, Wg2, Wg3,
      bg2.reshape(c1, 1), bg3.reshape(c3, 1))

    return (out.reshape(1, c3, nn, 1), cf.reshape(1, cp, nf, 1))


# speedup vs baseline: 1.1624x; 1.1624x over previous
import functools

import jax
import jax.numpy as jnp
from jax.experimental import pallas as pl
from jax.experimental.pallas import tpu as pltpu

_BLK = 2048
_EPS = 1e-5
_F32 = jnp.float32


def _cdiv(a, b):
    return (a + b - 1) // b


def _rowmask(c, n_true):
    row = pl.program_id(0) * _BLK + jax.lax.broadcasted_iota(
        jnp.int32, (_BLK, c), 0)
    return row < n_true


def _mat_stats_kernel(x_ref, wt_ref, s_ref, sum_ref, ssq_ref, *, n_true):
    s = jnp.dot(x_ref[...], wt_ref[...], preferred_element_type=_F32)
    s_ref[...] = s

    @pl.when(pl.program_id(0) == 0)
    def _init():
        sum_ref[...] = jnp.zeros_like(sum_ref)
        ssq_ref[...] = jnp.zeros_like(ssq_ref)

    sm = jnp.where(_rowmask(s.shape[1], n_true), s, 0.0)
    sum_ref[...] += jnp.sum(sm, axis=0, keepdims=True)
    ssq_ref[...] += jnp.sum(sm * sm, axis=0, keepdims=True)


def _norm_mat_stats_kernel(s_in_ref, sc_ref, sh_ref, wt_ref,
                           s_ref, sum_ref, ssq_ref, *, n_true):
    a = jnp.maximum(s_in_ref[...] * sc_ref[...] + sh_ref[...], 0.0)
    s = jnp.dot(a, wt_ref[...], preferred_element_type=_F32)
    s_ref[...] = s

    @pl.when(pl.program_id(0) == 0)
    def _init():
        sum_ref[...] = jnp.zeros_like(sum_ref)
        ssq_ref[...] = jnp.zeros_like(ssq_ref)

    sm = jnp.where(_rowmask(s.shape[1], n_true), s, 0.0)
    sum_ref[...] += jnp.sum(sm, axis=0, keepdims=True)
    ssq_ref[...] += jnp.sum(sm * sm, axis=0, keepdims=True)


def _matT_stats_kernel(x_ref, w_ref, s_ref, sum_ref, ssq_ref, *, n_true):
    s = jax.lax.dot_general(
        x_ref[...], w_ref[...], (((0,), (1,)), ((), ())),
        preferred_element_type=_F32)
    s_ref[...] = s

    @pl.when(pl.program_id(0) == 0)
    def _init():
        sum_ref[...] = jnp.zeros_like(sum_ref)
        ssq_ref[...] = jnp.zeros_like(ssq_ref)

    sm = jnp.where(_rowmask(s.shape[1], n_true), s, 0.0)
    sum_ref[...] += jnp.sum(sm, axis=0, keepdims=True)
    ssq_ref[...] += jnp.sum(sm * sm, axis=0, keepdims=True)


def _norm_relu_kernel(s_ref, sc_ref, sh_ref, o_ref):
    o_ref[...] = jnp.maximum(s_ref[...] * sc_ref[...] + sh_ref[...], 0.0)


def _norm_gram_kernel(s_ref, sc_ref, sh_ref, sum_ref, gram_ref, *, n_true):
    z = jnp.maximum(s_ref[...] * sc_ref[...] + sh_ref[...], 0.0)
    z = jnp.where(_rowmask(z.shape[1], n_true), z, 0.0)

    @pl.when(pl.program_id(0) == 0)
    def _init():
        sum_ref[...] = jnp.zeros_like(sum_ref)
        gram_ref[...] = jnp.zeros_like(gram_ref)

    sum_ref[...] += jnp.sum(z, axis=0, keepdims=True)
    gram_ref[...] += jax.lax.dot_general(
        z, z, (((0,), (0,)), ((), ())), preferred_element_type=_F32)


def _final_kernel(s3_ref, sc3_ref, sh3_ref, meanz_ref, cov_ref,
                  wg1t_ref, wg2t_ref, wg3t_ref, bg2_ref, bg3_ref,
                  out_ref, w1f_ref, sh4_ref):
    @pl.when(pl.program_id(0) == 0)
    def _fold():
        wg1t = wg1t_ref[...]
        m = jnp.dot(cov_ref[...], wg1t, preferred_element_type=_F32)
        var4 = jnp.sum(m * wg1t, axis=0, keepdims=True)
        mu4 = jnp.dot(meanz_ref[...], wg1t, preferred_element_type=_F32)
        inv4 = jax.lax.rsqrt(var4 + _EPS)
        w1f_ref[...] = wg1t * inv4
        sh4_ref[...] = -mu4 * inv4

    z = jnp.maximum(s3_ref[...] * sc3_ref[...] + sh3_ref[...], 0.0)
    h1 = jnp.dot(z, w1f_ref[...], preferred_element_type=_F32) + sh4_ref[...]
    h1 = jnp.where(h1 >= 0, h1, 0.01 * h1)
    h2 = jnp.dot(h1, wg2t_ref[...], preferred_element_type=_F32) + bg2_ref[...]
    h2 = jnp.where(h2 >= 0, h2, 0.01 * h2)
    out_ref[...] = (jnp.dot(h2, wg3t_ref[...], preferred_element_type=_F32)
                    + bg3_ref[...])


def _affine_from_stats(sm, sq, n):
    mu = sm / n
    var = sq / n - mu * mu
    inv = jax.lax.rsqrt(var + _EPS)
    return inv, -mu * inv


def _row_spec(c):
    return pl.BlockSpec((_BLK, c), lambda i: (i, 0))


def _full_spec(r, c):
    return pl.BlockSpec((r, c), lambda i: (0, 0))


def kernel(node_features, factor_features_0, nn_idx_0, etype_0,
           Wm0, bm0, Wm1, bm1, Wmp, bmp, Wg1, bg1, Wg2, bg2, Wg3, bg3):
    del nn_idx_0, etype_0, bm0, bm1, bmp, bg1
    xn = jnp.transpose(node_features.reshape(node_features.shape[1:3]))
    xf = factor_features_0.reshape(factor_features_0.shape[1:3])
    nn, cn_in = xn.shape
    nf = xf.shape[1]
    cm = Wm0.shape[0]
    cp = Wmp.shape[0]
    c1 = Wg1.shape[0]
    c3 = Wg3.shape[0]
    gn = _cdiv(nn, _BLK)
    gf = _cdiv(nf, _BLK)
    nnp = gn * _BLK
    nfp = gf * _BLK

    def mat_stats(x, wt, n, g, np_):
        ci, co = wt.shape
        return pl.pallas_call(
            functools.partial(_mat_stats_kernel, n_true=n),
            grid=(g,),
            in_specs=[_row_spec(ci), _full_spec(ci, co)],
            out_specs=[_row_spec(co), _full_spec(1, co), _full_spec(1, co)],
            out_shape=[jax.ShapeDtypeStruct((np_, co), _F32),
                       jax.ShapeDtypeStruct((1, co), _F32),
                       jax.ShapeDtypeStruct((1, co), _F32)],
        )(x, wt)

    def matT_stats(x, w, n, g, np_):
        co, ci = w.shape
        return pl.pallas_call(
            functools.partial(_matT_stats_kernel, n_true=n),
            grid=(g,),
            in_specs=[pl.BlockSpec((ci, _BLK), lambda i: (0, i)),
                      _full_spec(co, ci)],
            out_specs=[_row_spec(co), _full_spec(1, co), _full_spec(1, co)],
            out_shape=[jax.ShapeDtypeStruct((np_, co), _F32),
                       jax.ShapeDtypeStruct((1, co), _F32),
                       jax.ShapeDtypeStruct((1, co), _F32)],
        )(x, w)

    s1, sum1, ssq1 = mat_stats(xn, Wm0.T, nn, gn, nnp)
    s2, sum2, ssq2 = matT_stats(xf, Wm1, nf, gf, nfp)
    sc1, sh1 = _affine_from_stats(sum1, ssq1, nn)
    sc2, sh2 = _affine_from_stats(sum2, ssq2, nf)

    def norm_mat_stats(s_in, sc, sh, wt, n, g, np_):
        ci, co = wt.shape
        return pl.pallas_call(
            functools.partial(_norm_mat_stats_kernel, n_true=n),
            grid=(g,),
            in_specs=[_row_spec(ci), _full_spec(1, ci), _full_spec(1, ci),
                      _full_spec(ci, co)],
            out_specs=[_row_spec(co), _full_spec(1, co), _full_spec(1, co)],
            out_shape=[jax.ShapeDtypeStruct((np_, co), _F32),
                       jax.ShapeDtypeStruct((1, co), _F32),
                       jax.ShapeDtypeStruct((1, co), _F32)],
        )(s_in, sc, sh, wt)

    wmpt = Wmp.T
    s3n, sum3n, ssq3n = norm_mat_stats(s1, sc1, sh1, wmpt, nn, gn, nnp)
    s3f, sum3f, ssq3f = norm_mat_stats(s2, sc2, sh2, wmpt, nf, gf, nfp)
    sc3, sh3 = _affine_from_stats(sum3n + sum3f, ssq3n + ssq3f, nn + nf)

    cf = pl.pallas_call(
        _norm_relu_kernel,
        grid=(gf,),
        in_specs=[_row_spec(cp), _full_spec(1, cp), _full_spec(1, cp)],
        out_specs=_row_spec(cp),
        out_shape=jax.ShapeDtypeStruct((nf, cp), _F32),
    )(s3f, sc3, sh3)

    sumz, gram = pl.pallas_call(
        functools.partial(_norm_gram_kernel, n_true=nn),
        grid=(gn,),
        in_specs=[_row_spec(cp), _full_spec(1, cp), _full_spec(1, cp)],
        out_specs=[_full_spec(1, cp), _full_spec(cp, cp)],
        out_shape=[jax.ShapeDtypeStruct((1, cp), _F32),
                   jax.ShapeDtypeStruct((cp, cp), _F32)],
    )(s3n, sc3, sh3)
    meanz = sumz / nn
    cov = gram / nn - meanz.T * meanz

    out = pl.pallas_call(
        _final_kernel,
        grid=(gn,),
        in_specs=[_row_spec(cp), _full_spec(1, cp), _full_spec(1, cp),
                  _full_spec(1, cp), _full_spec(cp, cp),
                  _full_spec(cp, c1), _full_spec(c1, c1), _full_spec(c1, c3),
                  _full_spec(1, c1), _full_spec(1, c3)],
        out_specs=_row_spec(c3),
        out_shape=jax.ShapeDtypeStruct((nn, c3), _F32),
        scratch_shapes=[pltpu.VMEM((cp, c1), _F32),
                        pltpu.VMEM((1, c1), _F32)],
    )(s3n, sc3, sh3, meanz, cov, Wg1.T, Wg2.T, Wg3.T,
      bg2.reshape(1, c1), bg3.reshape(1, c3))

    return (jnp.transpose(out)[None, :, :, None],
            jnp.transpose(cf)[None, :, :, None])


# speedup vs baseline: 1.2952x; 1.1142x over previous
import functools

import jax
import jax.numpy as jnp
from jax.experimental import pallas as pl
from jax.experimental.pallas import tpu as pltpu

_BLK = 2048
_EPS = 1e-5
_F32 = jnp.float32
_BF16 = jnp.bfloat16


def _cdiv(a, b):
    return (a + b - 1) // b


def _rowmask(c, n_true):
    row = pl.program_id(0) * _BLK + jax.lax.broadcasted_iota(
        jnp.int32, (_BLK, c), 0)
    return row < n_true


def _accum_stats(s, sum_ref, ssq_ref, n_true):
    @pl.when(pl.program_id(0) == 0)
    def _init():
        sum_ref[...] = jnp.zeros_like(sum_ref)
        ssq_ref[...] = jnp.zeros_like(ssq_ref)

    sm = jnp.where(_rowmask(s.shape[1], n_true), s, 0.0)
    sum_ref[...] += jnp.sum(sm, axis=0, keepdims=True)
    ssq_ref[...] += jnp.sum(sm * sm, axis=0, keepdims=True)


def _mat_stats_kernel(x_ref, wt_ref, s_ref, sum_ref, ssq_ref, *, n_true):
    s = jnp.dot(x_ref[...].astype(_BF16), wt_ref[...],
                preferred_element_type=_F32)
    s_ref[...] = s.astype(_BF16)
    _accum_stats(s, sum_ref, ssq_ref, n_true)


def _matT_stats_kernel(x_ref, w_ref, s_ref, sum_ref, ssq_ref, *, n_true):
    s = jax.lax.dot_general(
        x_ref[...].astype(_BF16), w_ref[...], (((0,), (1,)), ((), ())),
        preferred_element_type=_F32)
    s_ref[...] = s.astype(_BF16)
    _accum_stats(s, sum_ref, ssq_ref, n_true)


def _norm_mat_stats_kernel(s_in_ref, sc_ref, sh_ref, wt_ref,
                           s_ref, sum_ref, ssq_ref, *, n_true):
    a = jnp.maximum(s_in_ref[...].astype(_F32) * sc_ref[...] + sh_ref[...],
                    0.0)
    s = jnp.dot(a.astype(_BF16), wt_ref[...], preferred_element_type=_F32)
    s_ref[...] = s.astype(_BF16)
    _accum_stats(s, sum_ref, ssq_ref, n_true)


def _norm_relu_kernel(s_ref, sc_ref, sh_ref, o_ref):
    o_ref[...] = jnp.maximum(
        s_ref[...].astype(_F32) * sc_ref[...] + sh_ref[...], 0.0)


def _norm_gram_kernel(s_ref, sc_ref, sh_ref, sum_ref, gram_ref, *, n_true):
    z = jnp.maximum(s_ref[...].astype(_F32) * sc_ref[...] + sh_ref[...], 0.0)
    z = jnp.where(_rowmask(z.shape[1], n_true), z, 0.0)

    @pl.when(pl.program_id(0) == 0)
    def _init():
        sum_ref[...] = jnp.zeros_like(sum_ref)
        gram_ref[...] = jnp.zeros_like(gram_ref)

    sum_ref[...] += jnp.sum(z, axis=0, keepdims=True)
    zb = z.astype(_BF16)
    gram_ref[...] += jax.lax.dot_general(
        zb, zb, (((0,), (0,)), ((), ())), preferred_element_type=_F32)


def _final_kernel(s3_ref, sc3_ref, sh3_ref, meanz_ref, cov_ref,
                  wg1t_ref, wg2t_ref, wg3t_ref, bg2_ref, bg3_ref,
                  out_ref, w1f_ref, sh4_ref):
    @pl.when(pl.program_id(0) == 0)
    def _fold():
        wg1t = wg1t_ref[...]
        m = jnp.dot(cov_ref[...], wg1t, preferred_element_type=_F32)
        var4 = jnp.sum(m * wg1t, axis=0, keepdims=True)
        mu4 = jnp.dot(meanz_ref[...], wg1t, preferred_element_type=_F32)
        inv4 = jax.lax.rsqrt(var4 + _EPS)
        w1f_ref[...] = (wg1t * inv4).astype(_BF16)
        sh4_ref[...] = -mu4 * inv4

    z = jnp.maximum(s3_ref[...].astype(_F32) * sc3_ref[...] + sh3_ref[...],
                    0.0)
    h1 = (jnp.dot(z.astype(_BF16), w1f_ref[...], preferred_element_type=_F32)
          + sh4_ref[...])
    h1 = jnp.where(h1 >= 0, h1, 0.01 * h1)
    h2 = (jnp.dot(h1.astype(_BF16), wg2t_ref[...],
                  preferred_element_type=_F32) + bg2_ref[...])
    h2 = jnp.where(h2 >= 0, h2, 0.01 * h2)
    out_ref[...] = (jnp.dot(h2.astype(_BF16), wg3t_ref[...],
                            preferred_element_type=_F32) + bg3_ref[...])


def _affine_from_stats(sm, sq, n):
    mu = sm / n
    var = sq / n - mu * mu
    inv = jax.lax.rsqrt(var + _EPS)
    return inv, -mu * inv


def _row_spec(c):
    return pl.BlockSpec((_BLK, c), lambda i: (i, 0))


def _full_spec(r, c):
    return pl.BlockSpec((r, c), lambda i: (0, 0))


def kernel(node_features, factor_features_0, nn_idx_0, etype_0,
           Wm0, bm0, Wm1, bm1, Wmp, bmp, Wg1, bg1, Wg2, bg2, Wg3, bg3):
    del nn_idx_0, etype_0, bm0, bm1, bmp, bg1
    xn = jnp.transpose(node_features.reshape(node_features.shape[1:3]))
    xf = factor_features_0.reshape(factor_features_0.shape[1:3])
    nn, cn_in = xn.shape
    nf = xf.shape[1]
    cm = Wm0.shape[0]
    cp = Wmp.shape[0]
    c1 = Wg1.shape[0]
    c3 = Wg3.shape[0]
    gn = _cdiv(nn, _BLK)
    gf = _cdiv(nf, _BLK)
    nnp = gn * _BLK
    nfp = gf * _BLK

    def mat_stats(x, wt, n, g, np_):
        ci, co = wt.shape
        return pl.pallas_call(
            functools.partial(_mat_stats_kernel, n_true=n),
            grid=(g,),
            in_specs=[_row_spec(ci), _full_spec(ci, co)],
            out_specs=[_row_spec(co), _full_spec(1, co), _full_spec(1, co)],
            out_shape=[jax.ShapeDtypeStruct((np_, co), _BF16),
                       jax.ShapeDtypeStruct((1, co), _F32),
                       jax.ShapeDtypeStruct((1, co), _F32)],
        )(x, wt)

    def matT_stats(x, w, n, g, np_):
        co, ci = w.shape
        return pl.pallas_call(
            functools.partial(_matT_stats_kernel, n_true=n),
            grid=(g,),
            in_specs=[pl.BlockSpec((ci, _BLK), lambda i: (0, i)),
                      _full_spec(co, ci)],
            out_specs=[_row_spec(co), _full_spec(1, co), _full_spec(1, co)],
            out_shape=[jax.ShapeDtypeStruct((np_, co), _BF16),
                       jax.ShapeDtypeStruct((1, co), _F32),
                       jax.ShapeDtypeStruct((1, co), _F32)],
        )(x, w)

    s1, sum1, ssq1 = mat_stats(xn, Wm0.T.astype(_BF16), nn, gn, nnp)
    s2, sum2, ssq2 = matT_stats(xf, Wm1.astype(_BF16), nf, gf, nfp)
    sc1, sh1 = _affine_from_stats(sum1, ssq1, nn)
    sc2, sh2 = _affine_from_stats(sum2, ssq2, nf)

    def norm_mat_stats(s_in, sc, sh, wt, n, g, np_):
        ci, co = wt.shape
        return pl.pallas_call(
            functools.partial(_norm_mat_stats_kernel, n_true=n),
            grid=(g,),
            in_specs=[_row_spec(ci), _full_spec(1, ci), _full_spec(1, ci),
                      _full_spec(ci, co)],
            out_specs=[_row_spec(co), _full_spec(1, co), _full_spec(1, co)],
            out_shape=[jax.ShapeDtypeStruct((np_, co), _BF16),
                       jax.ShapeDtypeStruct((1, co), _F32),
                       jax.ShapeDtypeStruct((1, co), _F32)],
        )(s_in, sc, sh, wt)

    wmpt = Wmp.T.astype(_BF16)
    s3n, sum3n, ssq3n = norm_mat_stats(s1, sc1, sh1, wmpt, nn, gn, nnp)
    s3f, sum3f, ssq3f = norm_mat_stats(s2, sc2, sh2, wmpt, nf, gf, nfp)
    sc3, sh3 = _affine_from_stats(sum3n + sum3f, ssq3n + ssq3f, nn + nf)

    cf = pl.pallas_call(
        _norm_relu_kernel,
        grid=(gf,),
        in_specs=[_row_spec(cp), _full_spec(1, cp), _full_spec(1, cp)],
        out_specs=_row_spec(cp),
        out_shape=jax.ShapeDtypeStruct((nf, cp), _F32),
    )(s3f, sc3, sh3)

    sumz, gram = pl.pallas_call(
        functools.partial(_norm_gram_kernel, n_true=nn),
        grid=(gn,),
        in_specs=[_row_spec(cp), _full_spec(1, cp), _full_spec(1, cp)],
        out_specs=[_full_spec(1, cp), _full_spec(cp, cp)],
        out_shape=[jax.ShapeDtypeStruct((1, cp), _F32),
                   jax.ShapeDtypeStruct((cp, cp), _F32)],
    )(s3n, sc3, sh3)
    meanz = sumz / nn
    cov = gram / nn - meanz.T * meanz

    out = pl.pallas_call(
        _final_kernel,
        grid=(gn,),
        in_specs=[_row_spec(cp), _full_spec(1, cp), _full_spec(1, cp),
                  _full_spec(1, cp), _full_spec(cp, cp),
                  _full_spec(cp, c1), _full_spec(c1, c1), _full_spec(c1, c3),
                  _full_spec(1, c1), _full_spec(1, c3)],
        out_specs=_row_spec(c3),
        out_shape=jax.ShapeDtypeStruct((nn, c3), _F32),
        scratch_shapes=[pltpu.VMEM((cp, c1), _BF16),
                        pltpu.VMEM((1, c1), _F32)],
    )(s3n, sc3, sh3, meanz, cov, Wg1.T, Wg2.T.astype(_BF16),
      Wg3.T.astype(_BF16), bg2.reshape(1, c1), bg3.reshape(1, c3))

    return (jnp.transpose(out)[None, :, :, None],
            jnp.transpose(cf)[None, :, :, None])


# speedup vs baseline: 1.5409x; 1.1897x over previous
import functools

import jax
import jax.numpy as jnp
from jax.experimental import pallas as pl
from jax.experimental.pallas import tpu as pltpu

_BLK = 4096
_EPS = 1e-5
_F32 = jnp.float32
_BF16 = jnp.bfloat16


def _cdiv(a, b):
    return (a + b - 1) // b


def _rowmask(c, n_true):
    row = pl.program_id(0) * _BLK + jax.lax.broadcasted_iota(
        jnp.int32, (_BLK, c), 0)
    return row < n_true


def _accum_stats(s, sum_ref, ssq_ref, n_true):
    @pl.when(pl.program_id(0) == 0)
    def _init():
        sum_ref[...] = jnp.zeros_like(sum_ref)
        ssq_ref[...] = jnp.zeros_like(ssq_ref)

    sm = jnp.where(_rowmask(s.shape[1], n_true), s, 0.0)
    sum_ref[...] += jnp.sum(sm, axis=0, keepdims=True)
    ssq_ref[...] += jnp.sum(sm * sm, axis=0, keepdims=True)


def _mat_stats_kernel(x_ref, wt_ref, s_ref, sum_ref, ssq_ref, *, n_true):
    s = jnp.dot(x_ref[...].astype(_BF16), wt_ref[...],
                preferred_element_type=_F32)
    s_ref[...] = s.astype(_BF16)
    _accum_stats(s, sum_ref, ssq_ref, n_true)


def _matT_stats_kernel(x_ref, w_ref, s_ref, sum_ref, ssq_ref, *, n_true):
    s = jax.lax.dot_general(
        x_ref[...].astype(_BF16), w_ref[...], (((0,), (1,)), ((), ())),
        preferred_element_type=_F32)
    s_ref[...] = s.astype(_BF16)
    _accum_stats(s, sum_ref, ssq_ref, n_true)


def _norm_mat_stats_kernel(s_in_ref, sc_ref, sh_ref, wt_ref,
                           s_ref, sum_ref, ssq_ref, *, n_true):
    a = jnp.maximum(s_in_ref[...].astype(_F32) * sc_ref[...] + sh_ref[...],
                    0.0)
    s = jnp.dot(a.astype(_BF16), wt_ref[...], preferred_element_type=_F32)
    s_ref[...] = s.astype(_BF16)
    _accum_stats(s, sum_ref, ssq_ref, n_true)


def _norm_relu_kernel(s_ref, sc_ref, sh_ref, o_ref):
    o_ref[...] = jnp.maximum(
        s_ref[...].astype(_F32) * sc_ref[...] + sh_ref[...], 0.0)


def _norm_gram_kernel(s_ref, sc_ref, sh_ref, sum_ref, gram_ref, *, n_true):
    z = jnp.maximum(s_ref[...].astype(_F32) * sc_ref[...] + sh_ref[...], 0.0)
    z = jnp.where(_rowmask(z.shape[1], n_true), z, 0.0)

    @pl.when(pl.program_id(0) == 0)
    def _init():
        sum_ref[...] = jnp.zeros_like(sum_ref)
        gram_ref[...] = jnp.zeros_like(gram_ref)

    sum_ref[...] += jnp.sum(z, axis=0, keepdims=True)
    zb = z.astype(_BF16)
    gram_ref[...] += jax.lax.dot_general(
        zb, zb, (((0,), (0,)), ((), ())), preferred_element_type=_F32)


def _final_kernel(s3_ref, sc3_ref, sh3_ref, meanz_ref, cov_ref,
                  wg1t_ref, wg2t_ref, wg3t_ref, bg2_ref, bg3_ref,
                  out_ref, w1f_ref, sh4_ref):
    @pl.when(pl.program_id(0) == 0)
    def _fold():
        wg1t = wg1t_ref[...]
        m = jnp.dot(cov_ref[...], wg1t, preferred_element_type=_F32)
        var4 = jnp.sum(m * wg1t, axis=0, keepdims=True)
        mu4 = jnp.dot(meanz_ref[...], wg1t, preferred_element_type=_F32)
        inv4 = jax.lax.rsqrt(var4 + _EPS)
        w1f_ref[...] = (wg1t * inv4).astype(_BF16)
        sh4_ref[...] = -mu4 * inv4

    z = jnp.maximum(s3_ref[...].astype(_F32) * sc3_ref[...] + sh3_ref[...],
                    0.0)
    h1 = (jnp.dot(z.astype(_BF16), w1f_ref[...], preferred_element_type=_F32)
          + sh4_ref[...])
    h1 = jnp.where(h1 >= 0, h1, 0.01 * h1)
    h2 = (jnp.dot(h1.astype(_BF16), wg2t_ref[...],
                  preferred_element_type=_F32) + bg2_ref[...])
    h2 = jnp.where(h2 >= 0, h2, 0.01 * h2)
    out_ref[...] = (jnp.dot(h2.astype(_BF16), wg3t_ref[...],
                            preferred_element_type=_F32) + bg3_ref[...])


def _affine_from_stats(sm, sq, n):
    mu = sm / n
    var = sq / n - mu * mu
    inv = jax.lax.rsqrt(var + _EPS)
    return inv, -mu * inv


def _row_spec(c):
    return pl.BlockSpec((_BLK, c), lambda i: (i, 0))


def _full_spec(r, c):
    return pl.BlockSpec((r, c), lambda i: (0, 0))


def kernel(node_features, factor_features_0, nn_idx_0, etype_0,
           Wm0, bm0, Wm1, bm1, Wmp, bmp, Wg1, bg1, Wg2, bg2, Wg3, bg3):
    del nn_idx_0, etype_0, bm0, bm1, bmp, bg1
    xn = jnp.transpose(node_features.reshape(node_features.shape[1:3]))
    xf = factor_features_0.reshape(factor_features_0.shape[1:3])
    nn, cn_in = xn.shape
    nf = xf.shape[1]
    cm = Wm0.shape[0]
    cp = Wmp.shape[0]
    c1 = Wg1.shape[0]
    c3 = Wg3.shape[0]
    gn = _cdiv(nn, _BLK)
    gf = _cdiv(nf, _BLK)
    nnp = gn * _BLK
    nfp = gf * _BLK

    def mat_stats(x, wt, n, g, np_):
        ci, co = wt.shape
        return pl.pallas_call(
            functools.partial(_mat_stats_kernel, n_true=n),
            grid=(g,),
            in_specs=[_row_spec(ci), _full_spec(ci, co)],
            out_specs=[_row_spec(co), _full_spec(1, co), _full_spec(1, co)],
            out_shape=[jax.ShapeDtypeStruct((np_, co), _BF16),
                       jax.ShapeDtypeStruct((1, co), _F32),
                       jax.ShapeDtypeStruct((1, co), _F32)],
        )(x, wt)

    def matT_stats(x, w, n, g, np_):
        co, ci = w.shape
        return pl.pallas_call(
            functools.partial(_matT_stats_kernel, n_true=n),
            grid=(g,),
            in_specs=[pl.BlockSpec((ci, _BLK), lambda i: (0, i)),
                      _full_spec(co, ci)],
            out_specs=[_row_spec(co), _full_spec(1, co), _full_spec(1, co)],
            out_shape=[jax.ShapeDtypeStruct((np_, co), _BF16),
                       jax.ShapeDtypeStruct((1, co), _F32),
                       jax.ShapeDtypeStruct((1, co), _F32)],
        )(x, w)

    s1, sum1, ssq1 = mat_stats(xn, Wm0.T.astype(_BF16), nn, gn, nnp)
    s2, sum2, ssq2 = matT_stats(xf, Wm1.astype(_BF16), nf, gf, nfp)
    sc1, sh1 = _affine_from_stats(sum1, ssq1, nn)
    sc2, sh2 = _affine_from_stats(sum2, ssq2, nf)

    def norm_mat_stats(s_in, sc, sh, wt, n, g, np_):
        ci, co = wt.shape
        return pl.pallas_call(
            functools.partial(_norm_mat_stats_kernel, n_true=n),
            grid=(g,),
            in_specs=[_row_spec(ci), _full_spec(1, ci), _full_spec(1, ci),
                      _full_spec(ci, co)],
            out_specs=[_row_spec(co), _full_spec(1, co), _full_spec(1, co)],
            out_shape=[jax.ShapeDtypeStruct((np_, co), _BF16),
                       jax.ShapeDtypeStruct((1, co), _F32),
                       jax.ShapeDtypeStruct((1, co), _F32)],
        )(s_in, sc, sh, wt)

    wmpt = Wmp.T.astype(_BF16)
    s3n, sum3n, ssq3n = norm_mat_stats(s1, sc1, sh1, wmpt, nn, gn, nnp)
    s3f, sum3f, ssq3f = norm_mat_stats(s2, sc2, sh2, wmpt, nf, gf, nfp)
    sc3, sh3 = _affine_from_stats(sum3n + sum3f, ssq3n + ssq3f, nn + nf)

    cf = pl.pallas_call(
        _norm_relu_kernel,
        grid=(gf,),
        in_specs=[_row_spec(cp), _full_spec(1, cp), _full_spec(1, cp)],
        out_specs=_row_spec(cp),
        out_shape=jax.ShapeDtypeStruct((nf, cp), _F32),
    )(s3f, sc3, sh3)

    sumz, gram = pl.pallas_call(
        functools.partial(_norm_gram_kernel, n_true=nn),
        grid=(gn,),
        in_specs=[_row_spec(cp), _full_spec(1, cp), _full_spec(1, cp)],
        out_specs=[_full_spec(1, cp), _full_spec(cp, cp)],
        out_shape=[jax.ShapeDtypeStruct((1, cp), _F32),
                   jax.ShapeDtypeStruct((cp, cp), _F32)],
    )(s3n, sc3, sh3)
    meanz = sumz / nn
    cov = gram / nn - meanz.T * meanz

    out = pl.pallas_call(
        _final_kernel,
        grid=(gn,),
        in_specs=[_row_spec(cp), _full_spec(1, cp), _full_spec(1, cp),
                  _full_spec(1, cp), _full_spec(cp, cp),
                  _full_spec(cp, c1), _full_spec(c1, c1), _full_spec(c1, c3),
                  _full_spec(1, c1), _full_spec(1, c3)],
        out_specs=_row_spec(c3),
        out_shape=jax.ShapeDtypeStruct((nn, c3), _F32),
        scratch_shapes=[pltpu.VMEM((cp, c1), _BF16),
                        pltpu.VMEM((1, c1), _F32)],
    )(s3n, sc3, sh3, meanz, cov, Wg1.T, Wg2.T.astype(_BF16),
      Wg3.T.astype(_BF16), bg2.reshape(1, c1), bg3.reshape(1, c3))

    return (jnp.transpose(out)[None, :, :, None],
            jnp.transpose(cf)[None, :, :, None])


# speedup vs baseline: 1.5637x; 1.0148x over previous
import functools

import jax
import jax.numpy as jnp
from jax.experimental import pallas as pl
from jax.experimental.pallas import tpu as pltpu

_BLK = 8192
_EPS = 1e-5
_F32 = jnp.float32
_BF16 = jnp.bfloat16


def _cdiv(a, b):
    return (a + b - 1) // b


def _rowmask(c, n_true):
    row = pl.program_id(0) * _BLK + jax.lax.broadcasted_iota(
        jnp.int32, (_BLK, c), 0)
    return row < n_true


def _accum_stats(s, sum_ref, ssq_ref, n_true):
    @pl.when(pl.program_id(0) == 0)
    def _init():
        sum_ref[...] = jnp.zeros_like(sum_ref)
        ssq_ref[...] = jnp.zeros_like(ssq_ref)

    sm = jnp.where(_rowmask(s.shape[1], n_true), s, 0.0)
    sum_ref[...] += jnp.sum(sm, axis=0, keepdims=True)
    ssq_ref[...] += jnp.sum(sm * sm, axis=0, keepdims=True)


def _mat_stats_kernel(x_ref, wt_ref, s_ref, sum_ref, ssq_ref, *, n_true):
    s = jnp.dot(x_ref[...].astype(_BF16), wt_ref[...],
                preferred_element_type=_F32)
    s_ref[...] = s.astype(_BF16)
    _accum_stats(s, sum_ref, ssq_ref, n_true)


def _matT_stats_kernel(x_ref, w_ref, s_ref, sum_ref, ssq_ref, *, n_true):
    s = jax.lax.dot_general(
        x_ref[...].astype(_BF16), w_ref[...], (((0,), (1,)), ((), ())),
        preferred_element_type=_F32)
    s_ref[...] = s.astype(_BF16)
    _accum_stats(s, sum_ref, ssq_ref, n_true)


def _norm_mat_stats_kernel(s_in_ref, sc_ref, sh_ref, wt_ref,
                           s_ref, sum_ref, ssq_ref, *, n_true):
    a = jnp.maximum(s_in_ref[...].astype(_F32) * sc_ref[...] + sh_ref[...],
                    0.0)
    s = jnp.dot(a.astype(_BF16), wt_ref[...], preferred_element_type=_F32)
    s_ref[...] = s.astype(_BF16)
    _accum_stats(s, sum_ref, ssq_ref, n_true)


def _norm_relu_kernel(s_ref, sc_ref, sh_ref, o_ref):
    o_ref[...] = jnp.maximum(
        s_ref[...].astype(_F32) * sc_ref[...] + sh_ref[...], 0.0)


def _norm_gram_kernel(s_ref, sc_ref, sh_ref, sum_ref, gram_ref, *, n_true):
    z = jnp.maximum(s_ref[...].astype(_F32) * sc_ref[...] + sh_ref[...], 0.0)
    z = jnp.where(_rowmask(z.shape[1], n_true), z, 0.0)

    @pl.when(pl.program_id(0) == 0)
    def _init():
        sum_ref[...] = jnp.zeros_like(sum_ref)
        gram_ref[...] = jnp.zeros_like(gram_ref)

    sum_ref[...] += jnp.sum(z, axis=0, keepdims=True)
    zb = z.astype(_BF16)
    gram_ref[...] += jax.lax.dot_general(
        zb, zb, (((0,), (0,)), ((), ())), preferred_element_type=_F32)


def _final_kernel(s3_ref, sc3_ref, sh3_ref, meanz_ref, cov_ref,
                  wg1t_ref, wg2t_ref, wg3t_ref, bg2_ref, bg3_ref,
                  out_ref, w1f_ref, sh4_ref):
    @pl.when(pl.program_id(0) == 0)
    def _fold():
        wg1t = wg1t_ref[...]
        m = jnp.dot(cov_ref[...], wg1t, preferred_element_type=_F32)
        var4 = jnp.sum(m * wg1t, axis=0, keepdims=True)
        mu4 = jnp.dot(meanz_ref[...], wg1t, preferred_element_type=_F32)
        inv4 = jax.lax.rsqrt(var4 + _EPS)
        w1f_ref[...] = (wg1t * inv4).astype(_BF16)
        sh4_ref[...] = -mu4 * inv4

    z = jnp.maximum(s3_ref[...].astype(_F32) * sc3_ref[...] + sh3_ref[...],
                    0.0)
    h1 = (jnp.dot(z.astype(_BF16), w1f_ref[...], preferred_element_type=_F32)
          + sh4_ref[...])
    h1 = jnp.where(h1 >= 0, h1, 0.01 * h1)
    h2 = (jnp.dot(h1.astype(_BF16), wg2t_ref[...],
                  preferred_element_type=_F32) + bg2_ref[...])
    h2 = jnp.where(h2 >= 0, h2, 0.01 * h2)
    out_ref[...] = (jnp.dot(h2.astype(_BF16), wg3t_ref[...],
                            preferred_element_type=_F32) + bg3_ref[...])


def _affine_from_stats(sm, sq, n):
    mu = sm / n
    var = sq / n - mu * mu
    inv = jax.lax.rsqrt(var + _EPS)
    return inv, -mu * inv


def _row_spec(c):
    return pl.BlockSpec((_BLK, c), lambda i: (i, 0))


def _full_spec(r, c):
    return pl.BlockSpec((r, c), lambda i: (0, 0))


def kernel(node_features, factor_features_0, nn_idx_0, etype_0,
           Wm0, bm0, Wm1, bm1, Wmp, bmp, Wg1, bg1, Wg2, bg2, Wg3, bg3):
    del nn_idx_0, etype_0, bm0, bm1, bmp, bg1
    xn = jnp.transpose(node_features.reshape(node_features.shape[1:3]))
    xf = factor_features_0.reshape(factor_features_0.shape[1:3])
    nn, cn_in = xn.shape
    nf = xf.shape[1]
    cm = Wm0.shape[0]
    cp = Wmp.shape[0]
    c1 = Wg1.shape[0]
    c3 = Wg3.shape[0]
    gn = _cdiv(nn, _BLK)
    gf = _cdiv(nf, _BLK)
    nnp = gn * _BLK
    nfp = gf * _BLK

    def mat_stats(x, wt, n, g, np_):
        ci, co = wt.shape
        return pl.pallas_call(
            functools.partial(_mat_stats_kernel, n_true=n),
            grid=(g,),
            in_specs=[_row_spec(ci), _full_spec(ci, co)],
            out_specs=[_row_spec(co), _full_spec(1, co), _full_spec(1, co)],
            out_shape=[jax.ShapeDtypeStruct((np_, co), _BF16),
                       jax.ShapeDtypeStruct((1, co), _F32),
                       jax.ShapeDtypeStruct((1, co), _F32)],
        )(x, wt)

    def matT_stats(x, w, n, g, np_):
        co, ci = w.shape
        return pl.pallas_call(
            functools.partial(_matT_stats_kernel, n_true=n),
            grid=(g,),
            in_specs=[pl.BlockSpec((ci, _BLK), lambda i: (0, i)),
                      _full_spec(co, ci)],
            out_specs=[_row_spec(co), _full_spec(1, co), _full_spec(1, co)],
            out_shape=[jax.ShapeDtypeStruct((np_, co), _BF16),
                       jax.ShapeDtypeStruct((1, co), _F32),
                       jax.ShapeDtypeStruct((1, co), _F32)],
        )(x, w)

    s1, sum1, ssq1 = mat_stats(xn, Wm0.T.astype(_BF16), nn, gn, nnp)
    s2, sum2, ssq2 = matT_stats(xf, Wm1.astype(_BF16), nf, gf, nfp)
    sc1, sh1 = _affine_from_stats(sum1, ssq1, nn)
    sc2, sh2 = _affine_from_stats(sum2, ssq2, nf)

    def norm_mat_stats(s_in, sc, sh, wt, n, g, np_):
        ci, co = wt.shape
        return pl.pallas_call(
            functools.partial(_norm_mat_stats_kernel, n_true=n),
            grid=(g,),
            in_specs=[_row_spec(ci), _full_spec(1, ci), _full_spec(1, ci),
                      _full_spec(ci, co)],
            out_specs=[_row_spec(co), _full_spec(1, co), _full_spec(1, co)],
            out_shape=[jax.ShapeDtypeStruct((np_, co), _BF16),
                       jax.ShapeDtypeStruct((1, co), _F32),
                       jax.ShapeDtypeStruct((1, co), _F32)],
        )(s_in, sc, sh, wt)

    wmpt = Wmp.T.astype(_BF16)
    s3n, sum3n, ssq3n = norm_mat_stats(s1, sc1, sh1, wmpt, nn, gn, nnp)
    s3f, sum3f, ssq3f = norm_mat_stats(s2, sc2, sh2, wmpt, nf, gf, nfp)
    sc3, sh3 = _affine_from_stats(sum3n + sum3f, ssq3n + ssq3f, nn + nf)

    cf = pl.pallas_call(
        _norm_relu_kernel,
        grid=(gf,),
        in_specs=[_row_spec(cp), _full_spec(1, cp), _full_spec(1, cp)],
        out_specs=_row_spec(cp),
        out_shape=jax.ShapeDtypeStruct((nf, cp), _F32),
    )(s3f, sc3, sh3)

    sumz, gram = pl.pallas_call(
        functools.partial(_norm_gram_kernel, n_true=nn),
        grid=(gn,),
        in_specs=[_row_spec(cp), _full_spec(1, cp), _full_spec(1, cp)],
        out_specs=[_full_spec(1, cp), _full_spec(cp, cp)],
        out_shape=[jax.ShapeDtypeStruct((1, cp), _F32),
                   jax.ShapeDtypeStruct((cp, cp), _F32)],
    )(s3n, sc3, sh3)
    meanz = sumz / nn
    cov = gram / nn - meanz.T * meanz

    out = pl.pallas_call(
        _final_kernel,
        grid=(gn,),
        in_specs=[_row_spec(cp), _full_spec(1, cp), _full_spec(1, cp),
                  _full_spec(1, cp), _full_spec(cp, cp),
                  _full_spec(cp, c1), _full_spec(c1, c1), _full_spec(c1, c3),
                  _full_spec(1, c1), _full_spec(1, c3)],
        out_specs=_row_spec(c3),
        out_shape=jax.ShapeDtypeStruct((nn, c3), _F32),
        scratch_shapes=[pltpu.VMEM((cp, c1), _BF16),
                        pltpu.VMEM((1, c1), _F32)],
    )(s3n, sc3, sh3, meanz, cov, Wg1.T, Wg2.T.astype(_BF16),
      Wg3.T.astype(_BF16), bg2.reshape(1, c1), bg3.reshape(1, c3))

    return (jnp.transpose(out)[None, :, :, None],
            jnp.transpose(cf)[None, :, :, None])


# speedup vs baseline: 2.3105x; 1.4776x over previous
import functools

import jax
import jax.numpy as jnp
from jax.experimental import pallas as pl
from jax.experimental.pallas import tpu as pltpu

_BLK = 4096
_EPS = 1e-5
_F32 = jnp.float32
_BF16 = jnp.bfloat16


def _blocks(n, blk):
    out = []
    st = 0
    while st < n:
        out.append((st, min(blk, n - st)))
        st += blk
    return out


def _leaky(x):
    return jnp.where(x >= 0, x, 0.01 * x)


def _mega_kernel(xn_ref, xf_ref, wm0t_ref, wm1_ref, wmpt_ref,
                 wg1t_ref, wg2t_ref, wg3t_ref, bg2_ref, bg3_ref,
                 out_ref, cf_ref,
                 s1_ref, s2_ref, s3n_ref, s3f_ref,
                 bufa_ref, bufb_ref, xfv_ref,
                 sem0, sem1, osem0, osem1, fsem,
                 *, nn, nf):
    nblk = _blocks(nn, _BLK)
    fblk = _blocks(nf, _BLK)
    bufs = (bufa_ref, bufb_ref)
    sems = (sem0, sem1)
    osems = (osem0, osem1)

    xf_copy = pltpu.make_async_copy(xf_ref, xfv_ref, fsem)
    xf_copy.start()

    def start_in_n(k):
        st, sz = nblk[k]
        pltpu.make_async_copy(
            xn_ref.at[pl.ds(st, sz), :],
            bufs[k % 2].at[pl.ds(0, sz), :], sems[k % 2]).start()

    def wait_in_n(k):
        st, sz = nblk[k]
        pltpu.make_async_copy(
            xn_ref.at[pl.ds(st, sz), :],
            bufs[k % 2].at[pl.ds(0, sz), :], sems[k % 2]).wait()

    sum1 = jnp.zeros((1, 64), _F32)
    ssq1 = jnp.zeros((1, 64), _F32)
    start_in_n(0)
    for k, (st, sz) in enumerate(nblk):
        if k + 1 < len(nblk):
            start_in_n(k + 1)
        wait_in_n(k)
        x = bufs[k % 2][pl.ds(0, sz), :]
        s = jnp.dot(x.astype(_BF16), wm0t_ref[...],
                    preferred_element_type=_F32)
        sum1 += jnp.sum(s, axis=0, keepdims=True)
        ssq1 += jnp.sum(s * s, axis=0, keepdims=True)
        s1_ref[pl.ds(st, sz), :] = s.astype(_BF16)

    xf_copy.wait()
    sum2 = jnp.zeros((1, 64), _F32)
    ssq2 = jnp.zeros((1, 64), _F32)
    for st, sz in fblk:
        x = xfv_ref[:, pl.ds(st, sz)]
        s = jax.lax.dot_general(
            x.astype(_BF16), wm1_ref[...], (((0,), (1,)), ((), ())),
            preferred_element_type=_F32)
        sum2 += jnp.sum(s, axis=0, keepdims=True)
        ssq2 += jnp.sum(s * s, axis=0, keepdims=True)
        s2_ref[pl.ds(st, sz), :] = s.astype(_BF16)

    def affine(sm, sq, n):
        mu = sm / n
        var = sq / n - mu * mu
        inv = jax.lax.rsqrt(var + _EPS)
        return inv, -mu * inv

    sc1, sh1 = affine(sum1, ssq1, nn)
    sc2, sh2 = affine(sum2, ssq2, nf)

    wmpt = wmpt_ref[...]
    sum3 = jnp.zeros((1, 128), _F32)
    ssq3 = jnp.zeros((1, 128), _F32)
    for st, sz in nblk:
        a = jnp.maximum(s1_ref[pl.ds(st, sz), :].astype(_F32) * sc1 + sh1,
                        0.0)
        s = jnp.dot(a.astype(_BF16), wmpt, preferred_element_type=_F32)
        sum3 += jnp.sum(s, axis=0, keepdims=True)
        ssq3 += jnp.sum(s * s, axis=0, keepdims=True)
        s3n_ref[pl.ds(st, sz), :] = s.astype(_BF16)
    for st, sz in fblk:
        a = jnp.maximum(s2_ref[pl.ds(st, sz), :].astype(_F32) * sc2 + sh2,
                        0.0)
        s = jnp.dot(a.astype(_BF16), wmpt, preferred_element_type=_F32)
        sum3 += jnp.sum(s, axis=0, keepdims=True)
        ssq3 += jnp.sum(s * s, axis=0, keepdims=True)
        s3f_ref[pl.ds(st, sz), :] = s.astype(_BF16)
    sc3, sh3 = affine(sum3, ssq3, nn + nf)

    for k, (st, sz) in enumerate(fblk):
        z = jnp.maximum(s3f_ref[pl.ds(st, sz), :].astype(_F32) * sc3 + sh3,
                        0.0)
        if k >= 2:
            pst, psz = fblk[k - 2]
            pltpu.make_async_copy(
                bufs[k % 2].at[pl.ds(0, psz), :],
                cf_ref.at[pl.ds(pst, psz), :], osems[k % 2]).wait()
        bufs[k % 2][pl.ds(0, sz), :] = z
        pltpu.make_async_copy(
            bufs[k % 2].at[pl.ds(0, sz), :],
            cf_ref.at[pl.ds(st, sz), :], osems[k % 2]).start()
    for k in range(max(0, len(fblk) - 2), len(fblk)):
        st, sz = fblk[k]
        pltpu.make_async_copy(
            bufs[k % 2].at[pl.ds(0, sz), :],
            cf_ref.at[pl.ds(st, sz), :], osems[k % 2]).wait()

    sumz = jnp.zeros((1, 128), _F32)
    gram = jnp.zeros((128, 128), _F32)
    for st, sz in nblk:
        z = jnp.maximum(s3n_ref[pl.ds(st, sz), :].astype(_F32) * sc3 + sh3,
                        0.0)
        sumz += jnp.sum(z, axis=0, keepdims=True)
        zb = z.astype(_BF16)
        gram += jax.lax.dot_general(
            zb, zb, (((0,), (0,)), ((), ())), preferred_element_type=_F32)
        s3n_ref[pl.ds(st, sz), :] = zb

    meanz = sumz / nn
    cov = gram / nn - meanz.T * meanz
    wg1t = wg1t_ref[...]
    m = jnp.dot(cov, wg1t, preferred_element_type=_F32)
    var4 = jnp.sum(m * wg1t, axis=0, keepdims=True)
    mu4 = jnp.dot(meanz, wg1t, preferred_element_type=_F32)
    inv4 = jax.lax.rsqrt(var4 + _EPS)
    w1f = (wg1t * inv4).astype(_BF16)
    sh4 = -mu4 * inv4

    wg2t = wg2t_ref[...]
    wg3t = wg3t_ref[...]
    bg2 = bg2_ref[...]
    bg3 = bg3_ref[...]
    for k, (st, sz) in enumerate(nblk):
        z = s3n_ref[pl.ds(st, sz), :]
        h1 = _leaky(jnp.dot(z, w1f, preferred_element_type=_F32) + sh4)
        h2 = _leaky(jnp.dot(h1.astype(_BF16), wg2t,
                            preferred_element_type=_F32) + bg2)
        o = (jnp.dot(h2.astype(_BF16), wg3t, preferred_element_type=_F32)
             + bg3)
        if k >= 2:
            pst, psz = nblk[k - 2]
            pltpu.make_async_copy(
                bufs[k % 2].at[pl.ds(0, psz), :],
                out_ref.at[pl.ds(pst, psz), :], osems[k % 2]).wait()
        bufs[k % 2][pl.ds(0, sz), :] = o
        pltpu.make_async_copy(
            bufs[k % 2].at[pl.ds(0, sz), :],
            out_ref.at[pl.ds(st, sz), :], osems[k % 2]).start()
    for k in range(max(0, len(nblk) - 2), len(nblk)):
        st, sz = nblk[k]
        pltpu.make_async_copy(
            bufs[k % 2].at[pl.ds(0, sz), :],
            out_ref.at[pl.ds(st, sz), :], osems[k % 2]).wait()


def kernel(node_features, factor_features_0, nn_idx_0, etype_0,
           Wm0, bm0, Wm1, bm1, Wmp, bmp, Wg1, bg1, Wg2, bg2, Wg3, bg3):
    del nn_idx_0, etype_0, bm0, bm1, bmp, bg1
    xn = jnp.transpose(node_features.reshape(node_features.shape[1:3]))
    xf = factor_features_0.reshape(factor_features_0.shape[1:3])
    nn, cn_in = xn.shape
    nf = xf.shape[1]
    cm = Wm0.shape[0]
    cp = Wmp.shape[0]
    c1 = Wg1.shape[0]
    c3 = Wg3.shape[0]

    any_spec = pl.BlockSpec(memory_space=pl.ANY)
    vmem_spec = pl.BlockSpec(memory_space=pltpu.VMEM)

    out, cf = pl.pallas_call(
        functools.partial(_mega_kernel, nn=nn, nf=nf),
        in_specs=[any_spec, any_spec] + [vmem_spec] * 8,
        out_specs=[any_spec, any_spec],
        out_shape=[jax.ShapeDtypeStruct((nn, c3), _F32),
                   jax.ShapeDtypeStruct((nf, cp), _F32)],
        scratch_shapes=[
            pltpu.VMEM((nn, cm), _BF16),
            pltpu.VMEM((nf, cm), _BF16),
            pltpu.VMEM((nn, cp), _BF16),
            pltpu.VMEM((nf, cp), _BF16),
            pltpu.VMEM((_BLK, cn_in), _F32),
            pltpu.VMEM((_BLK, cn_in), _F32),
            pltpu.VMEM((cm, nf), _F32),
            pltpu.SemaphoreType.DMA,
            pltpu.SemaphoreType.DMA,
            pltpu.SemaphoreType.DMA,
            pltpu.SemaphoreType.DMA,
            pltpu.SemaphoreType.DMA,
        ],
    )(xn, xf, Wm0.T.astype(_BF16), Wm1.astype(_BF16), Wmp.T.astype(_BF16),
      Wg1.T, Wg2.T.astype(_BF16), Wg3.T.astype(_BF16),
      bg2.reshape(1, c1), bg3.reshape(1, c3))

    return (jnp.transpose(out)[None, :, :, None],
            jnp.transpose(cf)[None, :, :, None])


# speedup vs baseline: 2.4933x; 1.0791x over previous
import functools

import jax
import jax.numpy as jnp
from jax.experimental import pallas as pl
from jax.experimental.pallas import tpu as pltpu

_BLK = 4096
_NBUF = 4
_EPS = 1e-5
_F32 = jnp.float32
_BF16 = jnp.bfloat16


def _blocks(n, blk):
    out = []
    st = 0
    while st < n:
        out.append((st, min(blk, n - st)))
        st += blk
    return out


def _leaky(x):
    return jnp.maximum(x, 0.01 * x)


def _mega_kernel(xn_ref, xf_ref, wm0t_ref, wm1_ref, wmpt_ref,
                 wg1t_ref, wg2t_ref, wg3t_ref, bg2_ref, bg3_ref,
                 out_ref, cf_ref,
                 s1_ref, s2_ref, s3n_ref, s3f_ref,
                 buf0, buf1, buf2, buf3, xfv_ref,
                 sem0, sem1, sem2, sem3, fsem0, fsem1,
                 *, nn, nf):
    nblk = _blocks(nn, _BLK)
    fblk = _blocks(nf, _BLK)
    bufs = (buf0, buf1, buf2, buf3)
    sems = (sem0, sem1, sem2, sem3)

    fc0 = pltpu.make_async_copy(xf_ref.at[pl.ds(0, 32), :],
                                xfv_ref.at[pl.ds(0, 32), :], fsem0)
    fc1 = pltpu.make_async_copy(xf_ref.at[pl.ds(32, 32), :],
                                xfv_ref.at[pl.ds(32, 32), :], fsem1)
    fc0.start()
    fc1.start()

    def in_n(k):
        st, sz = nblk[k]
        return pltpu.make_async_copy(
            xn_ref.at[pl.ds(st, sz), :],
            bufs[k % _NBUF].at[pl.ds(0, sz), :], sems[k % _NBUF])

    sum1 = jnp.zeros((1, 64), _F32)
    ssq1 = jnp.zeros((1, 64), _F32)
    for k in range(min(_NBUF, len(nblk))):
        in_n(k).start()
    for k, (st, sz) in enumerate(nblk):
        in_n(k).wait()
        x = bufs[k % _NBUF][pl.ds(0, sz), :]
        s = jnp.dot(x.astype(_BF16), wm0t_ref[...],
                    preferred_element_type=_F32)
        sum1 += jnp.sum(s, axis=0, keepdims=True)
        ssq1 += jnp.sum(s * s, axis=0, keepdims=True)
        s1_ref[pl.ds(st, sz), :] = s.astype(_BF16)
        if k + _NBUF < len(nblk):
            in_n(k + _NBUF).start()

    fc0.wait()
    fc1.wait()
    sum2 = jnp.zeros((1, 64), _F32)
    ssq2 = jnp.zeros((1, 64), _F32)
    for st, sz in fblk:
        x = xfv_ref[:, pl.ds(st, sz)]
        s = jax.lax.dot_general(
            x.astype(_BF16), wm1_ref[...], (((0,), (1,)), ((), ())),
            preferred_element_type=_F32)
        sum2 += jnp.sum(s, axis=0, keepdims=True)
        ssq2 += jnp.sum(s * s, axis=0, keepdims=True)
        s2_ref[pl.ds(st, sz), :] = s.astype(_BF16)

    def affine(sm, sq, n):
        mu = sm / n
        var = sq / n - mu * mu
        inv = jax.lax.rsqrt(var + _EPS)
        return inv, -mu * inv

    sc1, sh1 = affine(sum1, ssq1, nn)
    sc2, sh2 = affine(sum2, ssq2, nf)

    wmpt = wmpt_ref[...]
    sum3 = jnp.zeros((1, 128), _F32)
    ssq3 = jnp.zeros((1, 128), _F32)
    for st, sz in nblk:
        a = jnp.maximum(s1_ref[pl.ds(st, sz), :].astype(_F32) * sc1 + sh1,
                        0.0)
        s = jnp.dot(a.astype(_BF16), wmpt, preferred_element_type=_F32)
        sum3 += jnp.sum(s, axis=0, keepdims=True)
        ssq3 += jnp.sum(s * s, axis=0, keepdims=True)
        s3n_ref[pl.ds(st, sz), :] = s.astype(_BF16)
    for st, sz in fblk:
        a = jnp.maximum(s2_ref[pl.ds(st, sz), :].astype(_F32) * sc2 + sh2,
                        0.0)
        s = jnp.dot(a.astype(_BF16), wmpt, preferred_element_type=_F32)
        sum3 += jnp.sum(s, axis=0, keepdims=True)
        ssq3 += jnp.sum(s * s, axis=0, keepdims=True)
        s3f_ref[pl.ds(st, sz), :] = s.astype(_BF16)
    sc3, sh3 = affine(sum3, ssq3, nn + nf)

    def out_f(k):
        st, sz = fblk[k]
        return pltpu.make_async_copy(
            bufs[k % _NBUF].at[pl.ds(0, sz), :],
            cf_ref.at[pl.ds(st, sz), :], sems[k % _NBUF])

    for k, (st, sz) in enumerate(fblk):
        z = jnp.maximum(s3f_ref[pl.ds(st, sz), :].astype(_F32) * sc3 + sh3,
                        0.0)
        if k >= _NBUF:
            out_f(k - _NBUF).wait()
        bufs[k % _NBUF][pl.ds(0, sz), :] = z
        out_f(k).start()

    sumz = jnp.zeros((1, 128), _F32)
    gram = jnp.zeros((128, 128), _F32)
    for st, sz in nblk:
        z = jnp.maximum(s3n_ref[pl.ds(st, sz), :].astype(_F32) * sc3 + sh3,
                        0.0)
        sumz += jnp.sum(z, axis=0, keepdims=True)
        zb = z.astype(_BF16)
        gram += jax.lax.dot_general(
            zb, zb, (((0,), (0,)), ((), ())), preferred_element_type=_F32)
        s3n_ref[pl.ds(st, sz), :] = zb

    for k in range(max(0, len(fblk) - _NBUF), len(fblk)):
        out_f(k).wait()

    meanz = sumz / nn
    cov = gram / nn - meanz.T * meanz
    wg1t = wg1t_ref[...]
    m = jnp.dot(cov, wg1t, preferred_element_type=_F32)
    var4 = jnp.sum(m * wg1t, axis=0, keepdims=True)
    mu4 = jnp.dot(meanz, wg1t, preferred_element_type=_F32)
    inv4 = jax.lax.rsqrt(var4 + _EPS)
    w1f = (wg1t * inv4).astype(_BF16)
    sh4 = -mu4 * inv4

    def out_n(k):
        st, sz = nblk[k]
        return pltpu.make_async_copy(
            bufs[k % _NBUF].at[pl.ds(0, sz), :],
            out_ref.at[pl.ds(st, sz), :], sems[k % _NBUF])

    wg2t = wg2t_ref[...]
    wg3t = wg3t_ref[...]
    bg2 = bg2_ref[...]
    bg3 = bg3_ref[...]
    for k, (st, sz) in enumerate(nblk):
        z = s3n_ref[pl.ds(st, sz), :]
        h1 = _leaky(jnp.dot(z, w1f, preferred_element_type=_F32) + sh4)
        h2 = _leaky(jnp.dot(h1.astype(_BF16), wg2t,
                            preferred_element_type=_F32) + bg2)
        o = (jnp.dot(h2.astype(_BF16), wg3t, preferred_element_type=_F32)
             + bg3)
        if k >= _NBUF:
            out_n(k - _NBUF).wait()
        bufs[k % _NBUF][pl.ds(0, sz), :] = o
        out_n(k).start()
    for k in range(max(0, len(nblk) - _NBUF), len(nblk)):
        out_n(k).wait()


def kernel(node_features, factor_features_0, nn_idx_0, etype_0,
           Wm0, bm0, Wm1, bm1, Wmp, bmp, Wg1, bg1, Wg2, bg2, Wg3, bg3):
    del nn_idx_0, etype_0, bm0, bm1, bmp, bg1
    xn = jnp.transpose(node_features.reshape(node_features.shape[1:3]))
    xf = factor_features_0.reshape(factor_features_0.shape[1:3])
    nn, cn_in = xn.shape
    nf = xf.shape[1]
    cm = Wm0.shape[0]
    cp = Wmp.shape[0]
    c1 = Wg1.shape[0]
    c3 = Wg3.shape[0]

    any_spec = pl.BlockSpec(memory_space=pl.ANY)
    vmem_spec = pl.BlockSpec(memory_space=pltpu.VMEM)

    out, cf = pl.pallas_call(
        functools.partial(_mega_kernel, nn=nn, nf=nf),
        in_specs=[any_spec, any_spec] + [vmem_spec] * 8,
        out_specs=[any_spec, any_spec],
        out_shape=[jax.ShapeDtypeStruct((nn, c3), _F32),
                   jax.ShapeDtypeStruct((nf, cp), _F32)],
        scratch_shapes=[
            pltpu.VMEM((nn, cm), _BF16),
            pltpu.VMEM((nf, cm), _BF16),
            pltpu.VMEM((nn, cp), _BF16),
            pltpu.VMEM((nf, cp), _BF16),
            pltpu.VMEM((_BLK, cn_in), _F32),
            pltpu.VMEM((_BLK, cn_in), _F32),
            pltpu.VMEM((_BLK, cn_in), _F32),
            pltpu.VMEM((_BLK, cn_in), _F32),
            pltpu.VMEM((cm, nf), _F32),
            pltpu.SemaphoreType.DMA,
            pltpu.SemaphoreType.DMA,
            pltpu.SemaphoreType.DMA,
            pltpu.SemaphoreType.DMA,
            pltpu.SemaphoreType.DMA,
            pltpu.SemaphoreType.DMA,
        ],
        compiler_params=pltpu.CompilerParams(
            vmem_limit_bytes=64 * 1024 * 1024),
    )(xn, xf, Wm0.T.astype(_BF16), Wm1.astype(_BF16), Wmp.T.astype(_BF16),
      Wg1.T, Wg2.T.astype(_BF16), Wg3.T.astype(_BF16),
      bg2.reshape(1, c1), bg3.reshape(1, c3))

    return (jnp.transpose(out)[None, :, :, None],
            jnp.transpose(cf)[None, :, :, None])


# speedup vs baseline: 2.5164x; 1.0093x over previous
import functools

import jax
import jax.numpy as jnp
from jax.experimental import pallas as pl
from jax.experimental.pallas import tpu as pltpu

_BLK = 4096
_NBUF = 4
_EPS = 1e-5
_F32 = jnp.float32
_BF16 = jnp.bfloat16


def _blocks(n, blk):
    out = []
    st = 0
    while st < n:
        out.append((st, min(blk, n - st)))
        st += blk
    return out


def _leaky(x):
    return jnp.maximum(x, 0.01 * x)


def _colsum8(s):
    return jnp.sum(s.reshape(-1, 8, s.shape[1]), axis=0)


def _fold8(acc):
    return jnp.sum(acc, axis=0, keepdims=True)


def _mega_kernel(xn_ref, xf_ref, wm0t_ref, wm1_ref, wmpt_ref,
                 wg1t_ref, wg2t_ref, wg3t_ref, bg2_ref, bg3_ref,
                 out_ref, cf_ref,
                 s1_ref, s2_ref, s3n_ref, s3f_ref,
                 buf0, buf1, buf2, buf3, xfv_ref,
                 sem0, sem1, sem2, sem3, fsem0, fsem1,
                 *, nn, nf):
    nblk = _blocks(nn, _BLK)
    fblk = _blocks(nf, _BLK)
    bufs = (buf0, buf1, buf2, buf3)
    sems = (sem0, sem1, sem2, sem3)

    fc0 = pltpu.make_async_copy(xf_ref.at[pl.ds(0, 32), :],
                                xfv_ref.at[pl.ds(0, 32), :], fsem0)
    fc1 = pltpu.make_async_copy(xf_ref.at[pl.ds(32, 32), :],
                                xfv_ref.at[pl.ds(32, 32), :], fsem1)
    fc0.start()
    fc1.start()

    def in_n(k):
        st, sz = nblk[k]
        return pltpu.make_async_copy(
            xn_ref.at[pl.ds(st, sz), :],
            bufs[k % _NBUF].at[pl.ds(0, sz), :], sems[k % _NBUF])

    sum1 = jnp.zeros((8, 64), _F32)
    ssq1 = jnp.zeros((8, 64), _F32)
    for k in range(min(_NBUF, len(nblk))):
        in_n(k).start()
    for k, (st, sz) in enumerate(nblk):
        in_n(k).wait()
        x = bufs[k % _NBUF][pl.ds(0, sz), :]
        s = jnp.dot(x.astype(_BF16), wm0t_ref[...],
                    preferred_element_type=_F32)
        sum1 += _colsum8(s)
        ssq1 += _colsum8(s * s)
        s1_ref[pl.ds(st, sz), :] = s.astype(_BF16)
        if k + _NBUF < len(nblk):
            in_n(k + _NBUF).start()

    fc0.wait()
    fc1.wait()
    sum2 = jnp.zeros((8, 64), _F32)
    ssq2 = jnp.zeros((8, 64), _F32)
    for st, sz in fblk:
        x = xfv_ref[:, pl.ds(st, sz)]
        s = jax.lax.dot_general(
            x.astype(_BF16), wm1_ref[...], (((0,), (1,)), ((), ())),
            preferred_element_type=_F32)
        sum2 += _colsum8(s)
        ssq2 += _colsum8(s * s)
        s2_ref[pl.ds(st, sz), :] = s.astype(_BF16)

    def affine(sm8, sq8, n):
        mu = _fold8(sm8) / n
        var = _fold8(sq8) / n - mu * mu
        inv = jax.lax.rsqrt(var + _EPS)
        return inv.astype(_BF16), (-mu * inv).astype(_BF16)

    sc1, sh1 = affine(sum1, ssq1, nn)
    sc2, sh2 = affine(sum2, ssq2, nf)

    wmpt = wmpt_ref[...]
    zero_b = jnp.zeros((), _BF16)
    sum3 = jnp.zeros((8, 128), _F32)
    ssq3 = jnp.zeros((8, 128), _F32)
    for st, sz in nblk:
        a = jnp.maximum(s1_ref[pl.ds(st, sz), :] * sc1 + sh1, zero_b)
        s = jnp.dot(a, wmpt, preferred_element_type=_F32)
        sum3 += _colsum8(s)
        ssq3 += _colsum8(s * s)
        s3n_ref[pl.ds(st, sz), :] = s.astype(_BF16)
    for st, sz in fblk:
        a = jnp.maximum(s2_ref[pl.ds(st, sz), :] * sc2 + sh2, zero_b)
        s = jnp.dot(a, wmpt, preferred_element_type=_F32)
        sum3 += _colsum8(s)
        ssq3 += _colsum8(s * s)
        s3f_ref[pl.ds(st, sz), :] = s.astype(_BF16)
    sc3, sh3 = affine(sum3, ssq3, nn + nf)

    def out_f(k):
        st, sz = fblk[k]
        return pltpu.make_async_copy(
            bufs[k % _NBUF].at[pl.ds(0, sz), :],
            cf_ref.at[pl.ds(st, sz), :], sems[k % _NBUF])

    for k, (st, sz) in enumerate(fblk):
        z = jnp.maximum(s3f_ref[pl.ds(st, sz), :] * sc3 + sh3, zero_b)
        if k >= _NBUF:
            out_f(k - _NBUF).wait()
        bufs[k % _NBUF][pl.ds(0, sz), :] = z.astype(_F32)
        out_f(k).start()

    sumz8 = jnp.zeros((8, 128), _F32)
    gram = jnp.zeros((128, 128), _F32)
    for st, sz in nblk:
        zb = jnp.maximum(s3n_ref[pl.ds(st, sz), :] * sc3 + sh3, zero_b)
        sumz8 += _colsum8(zb.astype(_F32))
        gram += jax.lax.dot_general(
            zb, zb, (((0,), (0,)), ((), ())), preferred_element_type=_F32)
        s3n_ref[pl.ds(st, sz), :] = zb
    sumz = _fold8(sumz8)

    for k in range(max(0, len(fblk) - _NBUF), len(fblk)):
        out_f(k).wait()

    meanz = sumz / nn
    cov = gram / nn - meanz.T * meanz
    wg1t = wg1t_ref[...]
    m = jnp.dot(cov, wg1t, preferred_element_type=_F32)
    var4 = jnp.sum(m * wg1t, axis=0, keepdims=True)
    mu4 = jnp.dot(meanz, wg1t, preferred_element_type=_F32)
    inv4 = jax.lax.rsqrt(var4 + _EPS)
    w1f = (wg1t * inv4).astype(_BF16)
    sh4 = -mu4 * inv4

    def out_n(k):
        st, sz = nblk[k]
        return pltpu.make_async_copy(
            bufs[k % _NBUF].at[pl.ds(0, sz), :],
            out_ref.at[pl.ds(st, sz), :], sems[k % _NBUF])

    wg2t = wg2t_ref[...]
    wg3t = wg3t_ref[...]
    bg2 = bg2_ref[...]
    bg3 = bg3_ref[...]
    for k, (st, sz) in enumerate(nblk):
        z = s3n_ref[pl.ds(st, sz), :]
        h1 = _leaky((jnp.dot(z, w1f, preferred_element_type=_F32)
                     + sh4).astype(_BF16))
        h2 = _leaky((jnp.dot(h1, wg2t, preferred_element_type=_F32)
                     + bg2).astype(_BF16))
        o = (jnp.dot(h2, wg3t, preferred_element_type=_F32) + bg3)
        if k >= _NBUF:
            out_n(k - _NBUF).wait()
        bufs[k % _NBUF][pl.ds(0, sz), :] = o
        out_n(k).start()
    for k in range(max(0, len(nblk) - _NBUF), len(nblk)):
        out_n(k).wait()


def kernel(node_features, factor_features_0, nn_idx_0, etype_0,
           Wm0, bm0, Wm1, bm1, Wmp, bmp, Wg1, bg1, Wg2, bg2, Wg3, bg3):
    del nn_idx_0, etype_0, bm0, bm1, bmp, bg1
    xn = jnp.transpose(node_features.reshape(node_features.shape[1:3]))
    xf = factor_features_0.reshape(factor_features_0.shape[1:3])
    nn, cn_in = xn.shape
    nf = xf.shape[1]
    cm = Wm0.shape[0]
    cp = Wmp.shape[0]
    c1 = Wg1.shape[0]
    c3 = Wg3.shape[0]

    any_spec = pl.BlockSpec(memory_space=pl.ANY)
    vmem_spec = pl.BlockSpec(memory_space=pltpu.VMEM)

    out, cf = pl.pallas_call(
        functools.partial(_mega_kernel, nn=nn, nf=nf),
        in_specs=[any_spec, any_spec] + [vmem_spec] * 8,
        out_specs=[any_spec, any_spec],
        out_shape=[jax.ShapeDtypeStruct((nn, c3), _F32),
                   jax.ShapeDtypeStruct((nf, cp), _F32)],
        scratch_shapes=[
            pltpu.VMEM((nn, cm), _BF16),
            pltpu.VMEM((nf, cm), _BF16),
            pltpu.VMEM((nn, cp), _BF16),
            pltpu.VMEM((nf, cp), _BF16),
            pltpu.VMEM((_BLK, cn_in), _F32),
            pltpu.VMEM((_BLK, cn_in), _F32),
            pltpu.VMEM((_BLK, cn_in), _F32),
            pltpu.VMEM((_BLK, cn_in), _F32),
            pltpu.VMEM((cm, nf), _F32),
            pltpu.SemaphoreType.DMA,
            pltpu.SemaphoreType.DMA,
            pltpu.SemaphoreType.DMA,
            pltpu.SemaphoreType.DMA,
            pltpu.SemaphoreType.DMA,
            pltpu.SemaphoreType.DMA,
        ],
        compiler_params=pltpu.CompilerParams(
            vmem_limit_bytes=64 * 1024 * 1024),
    )(xn, xf, Wm0.T.astype(_BF16), Wm1.astype(_BF16), Wmp.T.astype(_BF16),
      Wg1.T, Wg2.T.astype(_BF16), Wg3.T.astype(_BF16),
      bg2.reshape(1, c1), bg3.reshape(1, c3))

    return (jnp.transpose(out)[None, :, :, None],
            jnp.transpose(cf)[None, :, :, None])


# speedup vs baseline: 2.5729x; 1.0224x over previous
import functools

import jax
import jax.numpy as jnp
from jax.experimental import pallas as pl
from jax.experimental.pallas import tpu as pltpu

_BLK = 4096
_NBUF = 4
_EPS = 1e-5
_F32 = jnp.float32
_BF16 = jnp.bfloat16


def _blocks(n, blk):
    out = []
    st = 0
    while st < n:
        out.append((st, min(blk, n - st)))
        st += blk
    return out


def _leaky(x):
    return jnp.maximum(x, 0.01 * x)


def _mega_kernel(xn_ref, xf_ref, wm0t_ref, wm1_ref, wmpt_ref,
                 wg1t_ref, wg2t_ref, wg3t_ref, bg2_ref, bg3_ref,
                 out_ref, cf_ref,
                 s1_ref, s2_ref, s3n_ref, s3f_ref,
                 buf0, buf1, xfv_ref,
                 sem0, sem1, sem2, sem3, fsem0, fsem1,
                 *, nn, nf):
    nblk = _blocks(nn, _BLK)
    fblk = _blocks(nf, _BLK)
    bufs = (buf0, buf1)
    sems = (sem0, sem1)

    fc0 = pltpu.make_async_copy(xf_ref.at[pl.ds(0, 32), :],
                                xfv_ref.at[pl.ds(0, 32), :], fsem0)
    fc1 = pltpu.make_async_copy(xf_ref.at[pl.ds(32, 32), :],
                                xfv_ref.at[pl.ds(32, 32), :], fsem1)
    fc0.start()
    fc1.start()

    cblk = _blocks(nn, 2 * _BLK)

    def in_n(k):
        st, sz = cblk[k]
        return pltpu.make_async_copy(
            xn_ref.at[pl.ds(st, sz), :],
            bufs[k % 2].at[pl.ds(0, sz), :], sems[k % 2])

    sum1 = jnp.zeros((1, 64), _F32)
    ssq1 = jnp.zeros((1, 64), _F32)
    in_n(0).start()
    if len(cblk) > 1:
        in_n(1).start()
    for k, (st, sz) in enumerate(cblk):
        in_n(k).wait()
        for sub, ssz in _blocks(sz, _BLK):
            x = bufs[k % 2][pl.ds(sub, ssz), :]
            s = jnp.dot(x.astype(_BF16), wm0t_ref[...],
                        preferred_element_type=_F32)
            sum1 += jnp.sum(s, axis=0, keepdims=True)
            ssq1 += jnp.sum(s * s, axis=0, keepdims=True)
            s1_ref[pl.ds(st + sub, ssz), :] = s.astype(_BF16)
        if k + 2 < len(cblk):
            in_n(k + 2).start()

    fc0.wait()
    fc1.wait()
    sum2 = jnp.zeros((1, 64), _F32)
    ssq2 = jnp.zeros((1, 64), _F32)
    for st, sz in fblk:
        x = xfv_ref[:, pl.ds(st, sz)]
        s = jax.lax.dot_general(
            x.astype(_BF16), wm1_ref[...], (((0,), (1,)), ((), ())),
            preferred_element_type=_F32)
        sum2 += jnp.sum(s, axis=0, keepdims=True)
        ssq2 += jnp.sum(s * s, axis=0, keepdims=True)
        s2_ref[pl.ds(st, sz), :] = s.astype(_BF16)

    def affine(sm, sq, n):
        mu = sm / n
        var = sq / n - mu * mu
        inv = jax.lax.rsqrt(var + _EPS)
        return inv, -mu * inv

    sc1, sh1 = affine(sum1, ssq1, nn)
    sc2, sh2 = affine(sum2, ssq2, nf)

    wmpt = wmpt_ref[...]
    sum3 = jnp.zeros((1, 128), _F32)
    ssq3 = jnp.zeros((1, 128), _F32)
    for st, sz in nblk:
        a = jnp.maximum(s1_ref[pl.ds(st, sz), :].astype(_F32) * sc1 + sh1,
                        0.0)
        s = jnp.dot(a.astype(_BF16), wmpt, preferred_element_type=_F32)
        sum3 += jnp.sum(s, axis=0, keepdims=True)
        ssq3 += jnp.sum(s * s, axis=0, keepdims=True)
        s3n_ref[pl.ds(st, sz), :] = s.astype(_BF16)
    for st, sz in fblk:
        a = jnp.maximum(s2_ref[pl.ds(st, sz), :].astype(_F32) * sc2 + sh2,
                        0.0)
        s = jnp.dot(a.astype(_BF16), wmpt, preferred_element_type=_F32)
        sum3 += jnp.sum(s, axis=0, keepdims=True)
        ssq3 += jnp.sum(s * s, axis=0, keepdims=True)
        s3f_ref[pl.ds(st, sz), :] = s.astype(_BF16)
    sc3, sh3 = affine(sum3, ssq3, nn + nf)

    fcblk = _blocks(nf, 2 * _BLK)

    def out_f(k):
        st, sz = fcblk[k]
        return pltpu.make_async_copy(
            bufs[k % 2].at[pl.ds(0, sz), :],
            cf_ref.at[pl.ds(st, sz), :], sems[k % 2])

    for k, (st, sz) in enumerate(fcblk):
        if k >= 2:
            out_f(k - 2).wait()
        for sub, ssz in _blocks(sz, _BLK):
            z = jnp.maximum(
                s3f_ref[pl.ds(st + sub, ssz), :].astype(_F32) * sc3 + sh3,
                0.0)
            bufs[k % 2][pl.ds(sub, ssz), :] = z
        out_f(k).start()

    sumz = jnp.zeros((1, 128), _F32)
    gram = jnp.zeros((128, 128), _F32)
    for st, sz in nblk:
        z = jnp.maximum(s3n_ref[pl.ds(st, sz), :].astype(_F32) * sc3 + sh3,
                        0.0)
        sumz += jnp.sum(z, axis=0, keepdims=True)
        zb = z.astype(_BF16)
        gram += jax.lax.dot_general(
            zb, zb, (((0,), (0,)), ((), ())), preferred_element_type=_F32)
        s3n_ref[pl.ds(st, sz), :] = zb

    for k in range(max(0, len(fcblk) - 2), len(fcblk)):
        out_f(k).wait()

    meanz = sumz / nn
    cov = gram / nn - meanz.T * meanz
    wg1t = wg1t_ref[...]
    m = jnp.dot(cov, wg1t, preferred_element_type=_F32)
    var4 = jnp.sum(m * wg1t, axis=0, keepdims=True)
    mu4 = jnp.dot(meanz, wg1t, preferred_element_type=_F32)
    inv4 = jax.lax.rsqrt(var4 + _EPS)
    w1f = (wg1t * inv4).astype(_BF16)
    sh4 = -mu4 * inv4

    def out_n(k):
        st, sz = cblk[k]
        return pltpu.make_async_copy(
            bufs[k % 2].at[pl.ds(0, sz), :],
            out_ref.at[pl.ds(st, sz), :], sems[k % 2])

    wg2t = wg2t_ref[...]
    wg3t = wg3t_ref[...]
    bg2 = bg2_ref[...]
    bg3 = bg3_ref[...]
    for k, (st, sz) in enumerate(cblk):
        if k >= 2:
            out_n(k - 2).wait()
        for sub, ssz in _blocks(sz, _BLK):
            z = s3n_ref[pl.ds(st + sub, ssz), :]
            h1 = _leaky(jnp.dot(z, w1f, preferred_element_type=_F32) + sh4)
            h2 = _leaky(jnp.dot(h1.astype(_BF16), wg2t,
                                preferred_element_type=_F32) + bg2)
            o = (jnp.dot(h2.astype(_BF16), wg3t,
                         preferred_element_type=_F32) + bg3)
            bufs[k % 2][pl.ds(sub, ssz), :] = o
        out_n(k).start()
    for k in range(max(0, len(cblk) - 2), len(cblk)):
        out_n(k).wait()


def kernel(node_features, factor_features_0, nn_idx_0, etype_0,
           Wm0, bm0, Wm1, bm1, Wmp, bmp, Wg1, bg1, Wg2, bg2, Wg3, bg3):
    del nn_idx_0, etype_0, bm0, bm1, bmp, bg1
    xn = jnp.transpose(node_features.reshape(node_features.shape[1:3]))
    xf = factor_features_0.reshape(factor_features_0.shape[1:3])
    nn, cn_in = xn.shape
    nf = xf.shape[1]
    cm = Wm0.shape[0]
    cp = Wmp.shape[0]
    c1 = Wg1.shape[0]
    c3 = Wg3.shape[0]

    any_spec = pl.BlockSpec(memory_space=pl.ANY)
    vmem_spec = pl.BlockSpec(memory_space=pltpu.VMEM)

    out, cf = pl.pallas_call(
        functools.partial(_mega_kernel, nn=nn, nf=nf),
        in_specs=[any_spec, any_spec] + [vmem_spec] * 8,
        out_specs=[any_spec, any_spec],
        out_shape=[jax.ShapeDtypeStruct((nn, c3), _F32),
                   jax.ShapeDtypeStruct((nf, cp), _F32)],
        scratch_shapes=[
            pltpu.VMEM((nn, cm), _BF16),
            pltpu.VMEM((nf, cm), _BF16),
            pltpu.VMEM((nn, cp), _BF16),
            pltpu.VMEM((nf, cp), _BF16),
            pltpu.VMEM((2 * _BLK, cn_in), _F32),
            pltpu.VMEM((2 * _BLK, cn_in), _F32),
            pltpu.VMEM((cm, nf), _F32),
            pltpu.SemaphoreType.DMA,
            pltpu.SemaphoreType.DMA,
            pltpu.SemaphoreType.DMA,
            pltpu.SemaphoreType.DMA,
            pltpu.SemaphoreType.DMA,
            pltpu.SemaphoreType.DMA,
        ],
        compiler_params=pltpu.CompilerParams(
            vmem_limit_bytes=64 * 1024 * 1024),
    )(xn, xf, Wm0.T.astype(_BF16), Wm1.astype(_BF16), Wmp.T.astype(_BF16),
      Wg1.T, Wg2.T.astype(_BF16), Wg3.T.astype(_BF16),
      bg2.reshape(1, c1), bg3.reshape(1, c3))

    return (jnp.transpose(out)[None, :, :, None],
            jnp.transpose(cf)[None, :, :, None])


# speedup vs baseline: 2.5761x; 1.0012x over previous
import functools

import jax
import jax.numpy as jnp
from jax.experimental import pallas as pl
from jax.experimental.pallas import tpu as pltpu

_BLK = 4096
_NBUF = 4
_EPS = 1e-5
_F32 = jnp.float32
_BF16 = jnp.bfloat16


def _blocks(n, blk):
    out = []
    st = 0
    while st < n:
        out.append((st, min(blk, n - st)))
        st += blk
    return out


def _leaky(x):
    return jnp.maximum(x, 0.01 * x)


def _mega_kernel(xn_ref, xf_ref, wm0t_ref, wm1_ref, wmpt_ref,
                 wg1t_ref, wg2t_ref, wg3t_ref, bg2_ref, bg3_ref,
                 out_ref, cf_ref,
                 s1_ref, s2_ref, s3n_ref, s3f_ref,
                 buf0, buf1, xfv_ref,
                 sem0, sem1, sem2, sem3, fsem0, fsem1,
                 *, nn, nf):
    nblk = _blocks(nn, _BLK)
    fblk = _blocks(nf, _BLK)
    bufs = (buf0, buf1)
    sems = (sem0, sem1)

    fc0 = pltpu.make_async_copy(xf_ref.at[pl.ds(0, 32), :],
                                xfv_ref.at[pl.ds(0, 32), :], fsem0)
    fc1 = pltpu.make_async_copy(xf_ref.at[pl.ds(32, 32), :],
                                xfv_ref.at[pl.ds(32, 32), :], fsem1)
    fc0.start()
    fc1.start()

    cblk = _blocks(nn, 2 * _BLK)

    def in_n(k):
        st, sz = cblk[k]
        return pltpu.make_async_copy(
            xn_ref.at[pl.ds(st, sz), :],
            bufs[k % 2].at[pl.ds(0, sz), :], sems[k % 2])

    sum1 = jnp.zeros((1, 64), _F32)
    ssq1 = jnp.zeros((1, 64), _F32)
    in_n(0).start()
    if len(cblk) > 1:
        in_n(1).start()
    for k, (st, sz) in enumerate(cblk):
        in_n(k).wait()
        for sub, ssz in _blocks(sz, _BLK):
            x = bufs[k % 2][pl.ds(sub, ssz), :]
            s = jnp.dot(x, wm0t_ref[...], preferred_element_type=_F32)
            sum1 += jnp.sum(s, axis=0, keepdims=True)
            ssq1 += jnp.sum(s * s, axis=0, keepdims=True)
            s1_ref[pl.ds(st + sub, ssz), :] = s.astype(_BF16)
        if k + 2 < len(cblk):
            in_n(k + 2).start()

    fc0.wait()
    fc1.wait()
    sum2 = jnp.zeros((1, 64), _F32)
    ssq2 = jnp.zeros((1, 64), _F32)
    for st, sz in fblk:
        x = xfv_ref[:, pl.ds(st, sz)]
        s = jax.lax.dot_general(
            x, wm1_ref[...], (((0,), (1,)), ((), ())),
            preferred_element_type=_F32)
        sum2 += jnp.sum(s, axis=0, keepdims=True)
        ssq2 += jnp.sum(s * s, axis=0, keepdims=True)
        s2_ref[pl.ds(st, sz), :] = s.astype(_BF16)

    def affine(sm, sq, n):
        mu = sm / n
        var = sq / n - mu * mu
        inv = jax.lax.rsqrt(var + _EPS)
        return inv, -mu * inv

    sc1, sh1 = affine(sum1, ssq1, nn)
    sc2, sh2 = affine(sum2, ssq2, nf)

    wmpt = wmpt_ref[...]
    sum3 = jnp.zeros((1, 128), _F32)
    ssq3 = jnp.zeros((1, 128), _F32)
    for st, sz in nblk:
        a = jnp.maximum(s1_ref[pl.ds(st, sz), :].astype(_F32) * sc1 + sh1,
                        0.0)
        s = jnp.dot(a.astype(_BF16), wmpt, preferred_element_type=_F32)
        sum3 += jnp.sum(s, axis=0, keepdims=True)
        ssq3 += jnp.sum(s * s, axis=0, keepdims=True)
        s3n_ref[pl.ds(st, sz), :] = s.astype(_BF16)
    for st, sz in fblk:
        a = jnp.maximum(s2_ref[pl.ds(st, sz), :].astype(_F32) * sc2 + sh2,
                        0.0)
        s = jnp.dot(a.astype(_BF16), wmpt, preferred_element_type=_F32)
        sum3 += jnp.sum(s, axis=0, keepdims=True)
        ssq3 += jnp.sum(s * s, axis=0, keepdims=True)
        s3f_ref[pl.ds(st, sz), :] = s.astype(_BF16)
    sc3, sh3 = affine(sum3, ssq3, nn + nf)

    fcblk = _blocks(nf, 2 * _BLK)

    def out_f(k):
        st, sz = fcblk[k]
        return pltpu.make_async_copy(
            bufs[k % 2].at[pl.ds(0, sz), :],
            cf_ref.at[pl.ds(st, sz), :], sems[k % 2])

    for k, (st, sz) in enumerate(fcblk):
        if k >= 2:
            out_f(k - 2).wait()
        for sub, ssz in _blocks(sz, _BLK):
            z = jnp.maximum(
                s3f_ref[pl.ds(st + sub, ssz), :].astype(_F32) * sc3 + sh3,
                0.0)
            bufs[k % 2][pl.ds(sub, ssz), :] = z
        out_f(k).start()

    sumz = jnp.zeros((1, 128), _F32)
    gram = jnp.zeros((128, 128), _F32)
    for st, sz in nblk:
        z = jnp.maximum(s3n_ref[pl.ds(st, sz), :].astype(_F32) * sc3 + sh3,
                        0.0)
        sumz += jnp.sum(z, axis=0, keepdims=True)
        zb = z.astype(_BF16)
        gram += jax.lax.dot_general(
            zb, zb, (((0,), (0,)), ((), ())), preferred_element_type=_F32)
        s3n_ref[pl.ds(st, sz), :] = zb

    for k in range(max(0, len(fcblk) - 2), len(fcblk)):
        out_f(k).wait()

    meanz = sumz / nn
    cov = gram / nn - meanz.T * meanz
    wg1t = wg1t_ref[...]
    m = jnp.dot(cov, wg1t, preferred_element_type=_F32)
    var4 = jnp.sum(m * wg1t, axis=0, keepdims=True)
    mu4 = jnp.dot(meanz, wg1t, preferred_element_type=_F32)
    inv4 = jax.lax.rsqrt(var4 + _EPS)
    w1f = (wg1t * inv4).astype(_BF16)
    sh4 = -mu4 * inv4

    def out_n(k):
        st, sz = cblk[k]
        return pltpu.make_async_copy(
            bufs[k % 2].at[pl.ds(0, sz), :],
            out_ref.at[pl.ds(st, sz), :], sems[k % 2])

    wg2t = wg2t_ref[...]
    wg3t = wg3t_ref[...]
    bg2 = bg2_ref[...]
    bg3 = bg3_ref[...]
    for k, (st, sz) in enumerate(cblk):
        if k >= 2:
            out_n(k - 2).wait()
        for sub, ssz in _blocks(sz, _BLK):
            z = s3n_ref[pl.ds(st + sub, ssz), :]
            h1 = _leaky(jnp.dot(z, w1f, preferred_element_type=_F32) + sh4)
            h2 = _leaky(jnp.dot(h1.astype(_BF16), wg2t,
                                preferred_element_type=_F32) + bg2)
            o = (jnp.dot(h2.astype(_BF16), wg3t,
                         preferred_element_type=_F32) + bg3)
            bufs[k % 2][pl.ds(sub, ssz), :] = o
        out_n(k).start()
    for k in range(max(0, len(cblk) - 2), len(cblk)):
        out_n(k).wait()


def kernel(node_features, factor_features_0, nn_idx_0, etype_0,
           Wm0, bm0, Wm1, bm1, Wmp, bmp, Wg1, bg1, Wg2, bg2, Wg3, bg3):
    del nn_idx_0, etype_0, bm0, bm1, bmp, bg1
    xn = jnp.transpose(node_features.reshape(node_features.shape[1:3]))
    xf = factor_features_0.reshape(factor_features_0.shape[1:3])
    nn, cn_in = xn.shape
    nf = xf.shape[1]
    cm = Wm0.shape[0]
    cp = Wmp.shape[0]
    c1 = Wg1.shape[0]
    c3 = Wg3.shape[0]

    any_spec = pl.BlockSpec(memory_space=pl.ANY)
    vmem_spec = pl.BlockSpec(memory_space=pltpu.VMEM)

    out, cf = pl.pallas_call(
        functools.partial(_mega_kernel, nn=nn, nf=nf),
        in_specs=[any_spec, any_spec] + [vmem_spec] * 8,
        out_specs=[any_spec, any_spec],
        out_shape=[jax.ShapeDtypeStruct((nn, c3), _F32),
                   jax.ShapeDtypeStruct((nf, cp), _F32)],
        scratch_shapes=[
            pltpu.VMEM((nn, cm), _BF16),
            pltpu.VMEM((nf, cm), _BF16),
            pltpu.VMEM((nn, cp), _BF16),
            pltpu.VMEM((nf, cp), _BF16),
            pltpu.VMEM((2 * _BLK, cn_in), _F32),
            pltpu.VMEM((2 * _BLK, cn_in), _F32),
            pltpu.VMEM((cm, nf), _F32),
            pltpu.SemaphoreType.DMA,
            pltpu.SemaphoreType.DMA,
            pltpu.SemaphoreType.DMA,
            pltpu.SemaphoreType.DMA,
            pltpu.SemaphoreType.DMA,
            pltpu.SemaphoreType.DMA,
        ],
        compiler_params=pltpu.CompilerParams(
            vmem_limit_bytes=64 * 1024 * 1024),
    )(xn, xf, Wm0.T, Wm1, Wmp.T.astype(_BF16),
      Wg1.T, Wg2.T.astype(_BF16), Wg3.T.astype(_BF16),
      bg2.reshape(1, c1), bg3.reshape(1, c3))

    return (jnp.transpose(out)[None, :, :, None],
            jnp.transpose(cf)[None, :, :, None])
